# baseline (speedup 1.0000x reference)
import functools

import jax
import jax.numpy as jnp
from jax import lax
from jax.experimental import pallas as pl
from jax.experimental.pallas import tpu as pltpu
from jax.experimental.pallas import tpu_sc as plsc

_NC = 2
_NS = 16
_NW = _NC * _NS


_NBUF = 4


def _make_agg(Np, H, E, with_deg):
  e_per_w = E // _NW
  CH = 80
  assert e_per_w % CH == 0 and E % _NW == 0
  rows_per_tile = Np // _NS
  assert rows_per_tile % 8 == 0
  n_ch = e_per_w // CH
  n_ch_p = ((n_ch + 7) // 8) * 8
  n_bgrp = n_ch_p // 8
  SUBS = 8 // _NBUF

  mesh = plsc.VectorSubcoreMesh(
      core_axis_name="c", subcore_axis_name="s",
      num_cores=_NC, num_subcores=_NS)

  out_type = [jax.ShapeDtypeStruct((_NC, Np, H), jnp.float32)]
  if with_deg:
    out_type.append(jax.ShapeDtypeStruct((_NC, Np), jnp.float32))

  scratch = (
      [pltpu.VMEM((16, CH), jnp.int32) for _ in range(3)]
      + [pltpu.VMEM((CH, H), jnp.float32) for _ in range(_NBUF)]
      + [pltpu.VMEM_SHARED((Np, H), jnp.float32)]
      + [pltpu.SemaphoreType.DMA for _ in range(3)]
      + [pltpu.SemaphoreType.DMA for _ in range(2 * _NBUF)]
  )
  if with_deg:
    scratch += (
        [pltpu.VMEM((CH,), jnp.float32),
         pltpu.VMEM_SHARED((Np,), jnp.float32)]
        + [pltpu.SemaphoreType.DMA for _ in range(_NBUF)]
    )

  assert n_bgrp >= 5
  n_trip = (n_bgrp - 4) // 3

  def body(x_hbm, ei_hbm, z2_hbm, z1_hbm, out_hbm, deg_hbm, *rest):
    idxg = rest[0:3]
    rows_v = rest[3:3 + _NBUF]
    acc_sh = rest[3 + _NBUF]
    isem = rest[4 + _NBUF:7 + _NBUF]
    gsem = rest[7 + _NBUF:7 + 2 * _NBUF]
    ssem = rest[7 + 2 * _NBUF:7 + 3 * _NBUF]
    if with_deg:
      tail = rest[7 + 3 * _NBUF:]
      ones_v, deg_sh = tail[0], tail[1]
      dsem = tail[2:2 + _NBUF]

    cid = lax.axis_index("c")
    sid = lax.axis_index("s")
    wid = sid * _NC + cid
    r0 = sid * rows_per_tile
    pltpu.sync_copy(z2_hbm.at[pl.ds(r0, rows_per_tile)],
                    acc_sh.at[pl.ds(r0, rows_per_tile)])
    if with_deg:
      for i in range(CH // 16):
        ones_v[pl.ds(16 * i, 16)] = jnp.full((16,), 1.0, jnp.float32)

      @pl.when(sid == 0)
      def _():
        pltpu.sync_copy(z1_hbm, deg_sh)
    plsc.subcore_barrier()

    def start_idx(g, j):
      off = pl.multiple_of(g * 16, 8)
      pltpu.async_copy(ei_hbm.at[wid, pl.ds(off, 16)], idxg[j], isem[j])

    def wait_idx(g, j):
      off = pl.multiple_of(g * 16, 8)
      pltpu.make_async_copy(ei_hbm.at[wid, pl.ds(off, 16)],
                            idxg[j], isem[j]).wait()

    def start_gather(j, s, b):
      pltpu.async_copy(x_hbm.at[idxg[j].at[2 * (s * _NBUF + b)]],
                       rows_v[b], gsem[b])

    def wait_gather(j, s, b):
      pltpu.make_async_copy(x_hbm.at[idxg[j].at[2 * (s * _NBUF + b)]],
                            rows_v[b], gsem[b]).wait()

    def start_scatter(j, s, b):
      pltpu.async_copy(rows_v[b],
                       acc_sh.at[idxg[j].at[2 * (s * _NBUF + b) + 1]],
                       ssem[b], add=True)
      if with_deg:
        pltpu.async_copy(ones_v,
                         deg_sh.at[idxg[j].at[2 * (s * _NBUF + b) + 1]],
                         dsem[b], add=True)

    def wait_scatter(j, s, b):
      pltpu.make_async_copy(
          rows_v[b], acc_sh.at[idxg[j].at[2 * (s * _NBUF + b) + 1]],
          ssem[b]).wait()
      if with_deg:
        pltpu.make_async_copy(
            ones_v, deg_sh.at[idxg[j].at[2 * (s * _NBUF + b) + 1]],
            dsem[b]).wait()

    def leg(g, jcur, jnxt, has_p2, has_p1):
      if has_p2:
        start_idx(g + 2, (jcur + 2) % 3)
      for s in range(SUBS):
        last = s == SUBS - 1
        if last and has_p1:
          wait_idx(g + 1, jnxt)
        for b in range(_NBUF):
          wait_gather(jcur, s, b)
          start_scatter(jcur, s, b)
        for b in range(_NBUF):
          wait_scatter(jcur, s, b)
          if not last:
            start_gather(jcur, s + 1, b)
          elif has_p1:
            start_gather(jnxt, 0, b)

    start_idx(0, 0)
    start_idx(1, 1)
    wait_idx(0, 0)
    for b in range(_NBUF):
      start_gather(0, 0, b)

    def triple(i, carry):
      for j in range(3):
        leg(3 * i + j, j, (j + 1) % 3, True, True)
      return carry

    lax.fori_loop(0, n_trip, triple, 0)
    for k in range(3 * n_trip, n_bgrp):
      leg(k, k % 3, (k + 1) % 3, k + 2 < n_bgrp, k + 1 < n_bgrp)

    plsc.subcore_barrier()
    pltpu.sync_copy(acc_sh.at[pl.ds(r0, rows_per_tile)],
                    out_hbm.at[cid, pl.ds(r0, rows_per_tile)])
    if with_deg:
      @pl.when(sid == 0)
      def _():
        pltpu.sync_copy(deg_sh, deg_hbm.at[cid])

  if with_deg:
    def body_wd(x_hbm, ei_hbm, z2_hbm, z1_hbm, out_hbm, deg_hbm, *rest):
      body(x_hbm, ei_hbm, z2_hbm, z1_hbm, out_hbm, deg_hbm, *rest)
    fn = pl.kernel(body_wd, out_type=out_type, mesh=mesh,
                   scratch_types=scratch)
    return lambda x, ei3, z2, z1: fn(x, ei3, z2, z1)
  else:
    def body_nd(x_hbm, ei_hbm, z2_hbm, out_hbm, *rest):
      body(x_hbm, ei_hbm, z2_hbm, None, out_hbm, None, *rest)
    fn = pl.kernel(body_nd, out_type=out_type[0], mesh=mesh,
                   scratch_types=scratch)
    return lambda x, ei3, z2: fn(x, ei3, z2)


def _combine(p, dpart, x, Wlt, Wrt, b, relu, pq=None):
  N, H = x.shape
  bm = 1024
  grid = (pl.cdiv(N, bm),)

  def compute_z(p_ref, d_ref, x_ref, wl_ref, wr_ref, b_ref):
    agg = p_ref[0] + p_ref[1]
    d = d_ref[0] + d_ref[1]
    scale = 1.0 / jnp.maximum(d, 1.0)
    aggn = agg * scale[:, None]
    acc = jnp.dot(aggn, wl_ref[...], preferred_element_type=jnp.float32)
    acc = acc + jnp.dot(x_ref[...], wr_ref[...],
                        preferred_element_type=jnp.float32)
    acc = acc + b_ref[...][None, :]
    if relu:
      acc = jnp.maximum(acc, 0.0)
    return acc

  base_specs = [
      pl.BlockSpec((_NC, bm, H), lambda i: (0, i, 0)),
      pl.BlockSpec((_NC, bm), lambda i: (0, i)),
      pl.BlockSpec((bm, H), lambda i: (i, 0)),
      pl.BlockSpec((H, H), lambda i: (0, 0)),
      pl.BlockSpec((H, H), lambda i: (0, 0)),
      pl.BlockSpec((H,), lambda i: (0,)),
  ]

  if pq is None:
    def body(p_ref, d_ref, x_ref, wl_ref, wr_ref, b_ref, o_ref):
      o_ref[...] = compute_z(p_ref, d_ref, x_ref, wl_ref, wr_ref, b_ref)

    return pl.pallas_call(
        body,
        grid=grid,
        in_specs=base_specs,
        out_specs=pl.BlockSpec((bm, H), lambda i: (i, 0)),
        out_shape=jax.ShapeDtypeStruct((N, H), jnp.float32),
    )(p, dpart, x, Wlt, Wrt, b)

  At, Bt = pq

  def body_pq(p_ref, d_ref, x_ref, wl_ref, wr_ref, b_ref, at_ref, bt_ref,
              po_ref, qo_ref):
    z = compute_z(p_ref, d_ref, x_ref, wl_ref, wr_ref, b_ref)
    po_ref[...] = jnp.dot(z, at_ref[...], preferred_element_type=jnp.float32)
    qo_ref[...] = jnp.dot(z, bt_ref[...], preferred_element_type=jnp.float32)

  return pl.pallas_call(
      body_pq,
      grid=grid,
      in_specs=base_specs + [
          pl.BlockSpec((H, H), lambda i: (0, 0)),
          pl.BlockSpec((H, H), lambda i: (0, 0)),
      ],
      out_specs=[
          pl.BlockSpec((bm, H), lambda i: (i, 0)),
          pl.BlockSpec((bm, H), lambda i: (i, 0)),
      ],
      out_shape=[
          jax.ShapeDtypeStruct((N, H), jnp.float32),
          jax.ShapeDtypeStruct((N, H), jnp.float32),
      ],
  )(p, dpart, x, Wlt, Wrt, b, At, Bt)


def _make_pair_gather(N, H, ELp):
  per_w = ELp // _NW
  CH = 112
  assert per_w % CH == 0 and ELp % _NW == 0
  n_ch = per_w // CH

  mesh = plsc.VectorSubcoreMesh(
      core_axis_name="c", subcore_axis_name="s",
      num_cores=_NC, num_subcores=_NS)

  NB = 4
  n_pairs = 2 * n_ch
  assert n_pairs % NB == 0
  n_grp = n_pairs // NB

  out_type = [
      jax.ShapeDtypeStruct((ELp, H), jnp.float32),
      jax.ShapeDtypeStruct((ELp, H), jnp.float32),
  ]
  scratch = (
      [pltpu.VMEM((CH,), jnp.int32) for _ in range(NB)]
      + [pltpu.VMEM((CH, H), jnp.float32) for _ in range(NB)]
      + [pltpu.SemaphoreType.DMA for _ in range(2 * NB)]
  )

  def body(p_hbm, q_hbm, idx_hbm, o1_hbm, o2_hbm, *rest):
    idx_v = rest[:NB]
    rows_v = rest[NB:2 * NB]
    gsem = rest[2 * NB:3 * NB]
    wsem = rest[3 * NB:4 * NB]
    cid = lax.axis_index("c")
    sid = lax.axis_index("s")
    wid = sid * _NC + cid
    base = wid * per_w
    base_r = wid * n_ch

    def chunk_idx(b, g):
      return g * (NB // 2) + b // 2

    def load_and_gather(b, g):
      c = chunk_idx(b, g)
      tab = p_hbm if b % 2 == 0 else q_hbm
      pltpu.sync_copy(idx_hbm.at[b % 2, base_r + c], idx_v[b])
      pltpu.async_copy(tab.at[idx_v[b]], rows_v[b], gsem[b])

    def wait_gather(b):
      tab = p_hbm if b % 2 == 0 else q_hbm
      pltpu.make_async_copy(tab.at[idx_v[b]], rows_v[b], gsem[b]).wait()

    def start_write(b, g):
      off = pl.multiple_of(base + chunk_idx(b, g) * CH, 8)
      dst_hbm = o1_hbm if b % 2 == 0 else o2_hbm
      pltpu.async_copy(rows_v[b], dst_hbm.at[pl.ds(off, CH)], wsem[b])

    def wait_write(b):
      pltpu.make_async_copy(
          rows_v[b], o1_hbm.at[pl.ds(0, CH)], wsem[b]).wait()

    for b in range(NB):
      load_and_gather(b, 0)

    def group(g, carry):
      for b in range(NB):
        wait_gather(b)
        start_write(b, g)
      for b in range(NB):
        wait_write(b)
        load_and_gather(b, g + 1)
      return carry

    lax.fori_loop(0, n_grp - 1, group, 0)
    for b in range(NB):
      wait_gather(b)
      start_write(b, n_grp - 1)
    for b in range(NB):
      wait_write(b)

  fn = pl.kernel(body, out_type=out_type, mesh=mesh, scratch_types=scratch)
  return fn


def _decoder(prow, qcol, d1b, d2, d2b):
  ELp, H = prow.shape
  bm = 2048
  assert ELp % bm == 0
  grid = (ELp // bm,)

  def body(zr_ref, zc_ref, bias_ref, d2_ref, d2b_ref, o_ref):
    t = zr_ref[...] + zc_ref[...] + bias_ref[...][None, :]
    t = jnp.maximum(t, 0.0)
    o_ref[...] = jnp.sum(t * d2_ref[...][None, :], axis=1) + d2b_ref[0]

  return pl.pallas_call(
      body,
      grid=grid,
      in_specs=[
          pl.BlockSpec((bm, H), lambda i: (i, 0)),
          pl.BlockSpec((bm, H), lambda i: (i, 0)),
          pl.BlockSpec((H,), lambda i: (0,)),
          pl.BlockSpec((H,), lambda i: (0,)),
          pl.BlockSpec(memory_space=pltpu.SMEM),
      ],
      out_specs=pl.BlockSpec((bm,), lambda i: (i,)),
      out_shape=jax.ShapeDtypeStruct((ELp,), jnp.float32),
  )(prow, qcol, d1b, d2, d2b)


def kernel(x, edge_index, edge_label_index,
           W1l, W1r, b1, W2l, W2r, b2, D1w, D1b, D2w, D2b):
  N, H = x.shape
  E = edge_index.shape[1]
  EL = edge_label_index.shape[1]

  Np = ((N + 1023) // 1024) * 1024
  zeros2d = jnp.zeros((Np, H), jnp.float32)
  zeros1d = jnp.zeros((Np,), jnp.float32)
  CH = 80
  n_ch = E // (_NW * CH)
  n_ch_p = ((n_ch + 7) // 8) * 8
  ei4 = edge_index.reshape(2, _NW, n_ch, CH)
  pad_dst = N + (jnp.arange(_NW * (n_ch_p - n_ch) * CH, dtype=jnp.int32)
                 % (Np - N)).reshape(1, _NW, n_ch_p - n_ch, CH)
  pad_src = jnp.zeros((1, _NW, n_ch_p - n_ch, CH), jnp.int32)
  ei4p = jnp.concatenate(
      [ei4, jnp.concatenate([pad_src, pad_dst], axis=0)], axis=2)
  ei4p = jnp.transpose(ei4p, (1, 2, 0, 3)).reshape(_NW, 2 * n_ch_p, CH)

  agg1 = _make_agg(Np, H, E, with_deg=True)
  agg2 = _make_agg(Np, H, E, with_deg=False)

  p1, dpart = agg1(x, ei4p, zeros2d, zeros1d)
  h = _combine(p1, dpart, x, W1l.T, W1r.T, b1, relu=True)
  p2 = agg2(h, ei4p, zeros2d)
  P, Q = _combine(p2, dpart, h, W2l.T, W2r.T, b2, relu=False,
                  pq=(D1w[:, :H].T, D1w[:, H:].T))

  chunk = _NW * 112
  ELp = ((EL + chunk - 1) // chunk) * chunk
  pad = ELp - EL
  eli3 = jnp.concatenate(
      [edge_label_index, jnp.zeros((2, pad), jnp.int32)],
      axis=1).reshape(2, ELp // 112, 112)
  prow, qcol = _make_pair_gather(N, H, ELp)(P, Q, eli3)
  out = _decoder(prow, qcol, D1b, D2w.reshape(H), D2b)
  return out[:EL]


# speedup vs baseline: 1.8053x; 1.8053x over previous
import functools

import jax
import jax.numpy as jnp
from jax import lax
from jax.experimental import pallas as pl
from jax.experimental.pallas import tpu as pltpu
from jax.experimental.pallas import tpu_sc as plsc

_NC = 2
_NS = 16
_NW = _NC * _NS


_NBUF = 4


def _make_agg(Np, H, E, with_deg):
  e_per_w = E // _NW
  CH = 80
  assert e_per_w % CH == 0 and E % _NW == 0
  rows_per_tile = Np // _NS
  assert rows_per_tile % 8 == 0
  n_ch = e_per_w // CH
  n_grp = n_ch // _NBUF
  rem = n_ch - n_grp * _NBUF

  mesh = plsc.VectorSubcoreMesh(
      core_axis_name="c", subcore_axis_name="s",
      num_cores=_NC, num_subcores=_NS)

  out_type = [jax.ShapeDtypeStruct((_NC, Np, H), jnp.float32)]
  if with_deg:
    out_type.append(jax.ShapeDtypeStruct((_NC, Np), jnp.float32))

  scratch = (
      [pltpu.VMEM((CH,), jnp.int32) for _ in range(_NBUF)]
      + [pltpu.VMEM((CH,), jnp.int32) for _ in range(_NBUF)]
      + [pltpu.VMEM((CH, H), jnp.float32) for _ in range(_NBUF)]
      + [pltpu.VMEM_SHARED((Np, H), jnp.float32)]
      + [pltpu.SemaphoreType.DMA for _ in range(2 * _NBUF)]
  )
  if with_deg:
    scratch += (
        [pltpu.VMEM((CH,), jnp.float32),
         pltpu.VMEM_SHARED((Np,), jnp.float32)]
        + [pltpu.SemaphoreType.DMA for _ in range(_NBUF)]
    )

  def body(x_hbm, ei_hbm, z2_hbm, z1_hbm, out_hbm, deg_hbm, *rest):
    src_v = rest[:_NBUF]
    dst_v = rest[_NBUF:2 * _NBUF]
    rows_v = rest[2 * _NBUF:3 * _NBUF]
    acc_sh = rest[3 * _NBUF]
    gsem = rest[3 * _NBUF + 1:3 * _NBUF + 1 + _NBUF]
    ssem = rest[3 * _NBUF + 1 + _NBUF:3 * _NBUF + 1 + 2 * _NBUF]
    if with_deg:
      tail = rest[3 * _NBUF + 1 + 2 * _NBUF:]
      ones_v, deg_sh = tail[0], tail[1]
      dsem = tail[2:2 + _NBUF]

    cid = lax.axis_index("c")
    sid = lax.axis_index("s")
    wid = sid * _NC + cid
    r0 = sid * rows_per_tile
    pltpu.sync_copy(z2_hbm.at[pl.ds(r0, rows_per_tile)],
                    acc_sh.at[pl.ds(r0, rows_per_tile)])
    if with_deg:
      for i in range(CH // 16):
        ones_v[pl.ds(16 * i, 16)] = jnp.full((16,), 1.0, jnp.float32)

      @pl.when(sid == 0)
      def _():
        pltpu.sync_copy(z1_hbm, deg_sh)
    plsc.subcore_barrier()

    base = wid * n_ch

    def load_and_gather(b, c):
      r = base + c
      pltpu.sync_copy(ei_hbm.at[0, r], src_v[b])
      pltpu.sync_copy(ei_hbm.at[1, r], dst_v[b])
      pltpu.async_copy(x_hbm.at[src_v[b]], rows_v[b], gsem[b])

    def start_scatter(b):
      pltpu.async_copy(rows_v[b], acc_sh.at[dst_v[b]], ssem[b], add=True)
      if with_deg:
        pltpu.async_copy(ones_v, deg_sh.at[dst_v[b]], dsem[b], add=True)

    def wait_gather(b):
      pltpu.make_async_copy(x_hbm.at[src_v[b]], rows_v[b], gsem[b]).wait()

    def wait_scatter(b):
      pltpu.make_async_copy(rows_v[b], acc_sh.at[dst_v[b]], ssem[b]).wait()
      if with_deg:
        pltpu.make_async_copy(ones_v, deg_sh.at[dst_v[b]], dsem[b]).wait()

    for b in range(_NBUF):
      load_and_gather(b, b)

    def group(g, carry):
      for b in range(_NBUF):
        wait_gather(b)
        start_scatter(b)
      for b in range(_NBUF):
        wait_scatter(b)
        load_and_gather(b, (g + 1) * _NBUF + b)
      return carry

    lax.fori_loop(0, n_grp - 1, group, 0)
    for b in range(_NBUF):
      wait_gather(b)
      start_scatter(b)
    for b in range(_NBUF):
      wait_scatter(b)
    for r in range(rem):
      load_and_gather(r, n_grp * _NBUF + r)
      wait_gather(r)
      start_scatter(r)
      wait_scatter(r)

    plsc.subcore_barrier()
    pltpu.sync_copy(acc_sh.at[pl.ds(r0, rows_per_tile)],
                    out_hbm.at[cid, pl.ds(r0, rows_per_tile)])
    if with_deg:
      @pl.when(sid == 0)
      def _():
        pltpu.sync_copy(deg_sh, deg_hbm.at[cid])

  if with_deg:
    def body_wd(x_hbm, ei_hbm, z2_hbm, z1_hbm, out_hbm, deg_hbm, *rest):
      body(x_hbm, ei_hbm, z2_hbm, z1_hbm, out_hbm, deg_hbm, *rest)
    fn = pl.kernel(body_wd, out_type=out_type, mesh=mesh,
                   scratch_types=scratch)
    return lambda x, ei3, z2, z1: fn(x, ei3, z2, z1)
  else:
    def body_nd(x_hbm, ei_hbm, z2_hbm, out_hbm, *rest):
      body(x_hbm, ei_hbm, z2_hbm, None, out_hbm, None, *rest)
    fn = pl.kernel(body_nd, out_type=out_type[0], mesh=mesh,
                   scratch_types=scratch)
    return lambda x, ei3, z2: fn(x, ei3, z2)


def _combine(p, dpart, x, Wlt, Wrt, b, relu, pq=None):
  N, H = x.shape
  bm = 1024
  grid = (pl.cdiv(N, bm),)

  def compute_z(p_ref, d_ref, x_ref, wl_ref, wr_ref, b_ref):
    agg = p_ref[0] + p_ref[1]
    d = d_ref[0] + d_ref[1]
    scale = 1.0 / jnp.maximum(d, 1.0)
    aggn = agg * scale[:, None]
    acc = jnp.dot(aggn, wl_ref[...], preferred_element_type=jnp.float32)
    acc = acc + jnp.dot(x_ref[...], wr_ref[...],
                        preferred_element_type=jnp.float32)
    acc = acc + b_ref[...][None, :]
    if relu:
      acc = jnp.maximum(acc, 0.0)
    return acc

  base_specs = [
      pl.BlockSpec((_NC, bm, H), lambda i: (0, i, 0)),
      pl.BlockSpec((_NC, bm), lambda i: (0, i)),
      pl.BlockSpec((bm, H), lambda i: (i, 0)),
      pl.BlockSpec((H, H), lambda i: (0, 0)),
      pl.BlockSpec((H, H), lambda i: (0, 0)),
      pl.BlockSpec((H,), lambda i: (0,)),
  ]

  if pq is None:
    def body(p_ref, d_ref, x_ref, wl_ref, wr_ref, b_ref, o_ref):
      o_ref[...] = compute_z(p_ref, d_ref, x_ref, wl_ref, wr_ref, b_ref)

    return pl.pallas_call(
        body,
        grid=grid,
        in_specs=base_specs,
        out_specs=pl.BlockSpec((bm, H), lambda i: (i, 0)),
        out_shape=jax.ShapeDtypeStruct((N, H), jnp.float32),
    )(p, dpart, x, Wlt, Wrt, b)

  At, Bt = pq

  def body_pq(p_ref, d_ref, x_ref, wl_ref, wr_ref, b_ref, at_ref, bt_ref,
              po_ref, qo_ref):
    z = compute_z(p_ref, d_ref, x_ref, wl_ref, wr_ref, b_ref)
    po_ref[...] = jnp.dot(z, at_ref[...], preferred_element_type=jnp.float32)
    qo_ref[...] = jnp.dot(z, bt_ref[...], preferred_element_type=jnp.float32)

  return pl.pallas_call(
      body_pq,
      grid=grid,
      in_specs=base_specs + [
          pl.BlockSpec((H, H), lambda i: (0, 0)),
          pl.BlockSpec((H, H), lambda i: (0, 0)),
      ],
      out_specs=[
          pl.BlockSpec((bm, H), lambda i: (i, 0)),
          pl.BlockSpec((bm, H), lambda i: (i, 0)),
      ],
      out_shape=[
          jax.ShapeDtypeStruct((N, H), jnp.float32),
          jax.ShapeDtypeStruct((N, H), jnp.float32),
      ],
  )(p, dpart, x, Wlt, Wrt, b, At, Bt)


def _make_pair_gather(N, H, ELp):
  per_w = ELp // _NW
  CH = 112
  assert per_w % CH == 0 and ELp % _NW == 0
  n_ch = per_w // CH

  mesh = plsc.VectorSubcoreMesh(
      core_axis_name="c", subcore_axis_name="s",
      num_cores=_NC, num_subcores=_NS)

  NB = 4
  n_pairs = 2 * n_ch
  assert n_pairs % NB == 0
  n_grp = n_pairs // NB

  out_type = [
      jax.ShapeDtypeStruct((ELp, H), jnp.float32),
      jax.ShapeDtypeStruct((ELp, H), jnp.float32),
  ]
  scratch = (
      [pltpu.VMEM((CH,), jnp.int32) for _ in range(NB)]
      + [pltpu.VMEM((CH, H), jnp.float32) for _ in range(NB)]
      + [pltpu.SemaphoreType.DMA for _ in range(2 * NB)]
  )

  def body(p_hbm, q_hbm, idx_hbm, o1_hbm, o2_hbm, *rest):
    idx_v = rest[:NB]
    rows_v = rest[NB:2 * NB]
    gsem = rest[2 * NB:3 * NB]
    wsem = rest[3 * NB:4 * NB]
    cid = lax.axis_index("c")
    sid = lax.axis_index("s")
    wid = sid * _NC + cid
    base = wid * per_w
    base_r = wid * n_ch

    def chunk_idx(b, g):
      return g * (NB // 2) + b // 2

    def load_and_gather(b, g):
      c = chunk_idx(b, g)
      tab = p_hbm if b % 2 == 0 else q_hbm
      pltpu.sync_copy(idx_hbm.at[b % 2, base_r + c], idx_v[b])
      pltpu.async_copy(tab.at[idx_v[b]], rows_v[b], gsem[b])

    def wait_gather(b):
      tab = p_hbm if b % 2 == 0 else q_hbm
      pltpu.make_async_copy(tab.at[idx_v[b]], rows_v[b], gsem[b]).wait()

    def start_write(b, g):
      off = pl.multiple_of(base + chunk_idx(b, g) * CH, 8)
      dst_hbm = o1_hbm if b % 2 == 0 else o2_hbm
      pltpu.async_copy(rows_v[b], dst_hbm.at[pl.ds(off, CH)], wsem[b])

    def wait_write(b):
      pltpu.make_async_copy(
          rows_v[b], o1_hbm.at[pl.ds(0, CH)], wsem[b]).wait()

    for b in range(NB):
      load_and_gather(b, 0)

    def group(g, carry):
      for b in range(NB):
        wait_gather(b)
        start_write(b, g)
      for b in range(NB):
        wait_write(b)
        load_and_gather(b, g + 1)
      return carry

    lax.fori_loop(0, n_grp - 1, group, 0)
    for b in range(NB):
      wait_gather(b)
      start_write(b, n_grp - 1)
    for b in range(NB):
      wait_write(b)

  fn = pl.kernel(body, out_type=out_type, mesh=mesh, scratch_types=scratch)
  return fn


def _decoder(prow, qcol, d1b, d2, d2b):
  ELp, H = prow.shape
  bm = 2048
  assert ELp % bm == 0
  grid = (ELp // bm,)

  def body(zr_ref, zc_ref, bias_ref, d2_ref, d2b_ref, o_ref):
    t = zr_ref[...] + zc_ref[...] + bias_ref[...][None, :]
    t = jnp.maximum(t, 0.0)
    o_ref[...] = jnp.sum(t * d2_ref[...][None, :], axis=1) + d2b_ref[0]

  return pl.pallas_call(
      body,
      grid=grid,
      in_specs=[
          pl.BlockSpec((bm, H), lambda i: (i, 0)),
          pl.BlockSpec((bm, H), lambda i: (i, 0)),
          pl.BlockSpec((H,), lambda i: (0,)),
          pl.BlockSpec((H,), lambda i: (0,)),
          pl.BlockSpec(memory_space=pltpu.SMEM),
      ],
      out_specs=pl.BlockSpec((bm,), lambda i: (i,)),
      out_shape=jax.ShapeDtypeStruct((ELp,), jnp.float32),
  )(prow, qcol, d1b, d2, d2b)


def kernel(x, edge_index, edge_label_index,
           W1l, W1r, b1, W2l, W2r, b2, D1w, D1b, D2w, D2b):
  N, H = x.shape
  E = edge_index.shape[1]
  EL = edge_label_index.shape[1]

  Np = ((N + 1023) // 1024) * 1024
  zeros2d = jnp.zeros((Np, H), jnp.float32)
  zeros1d = jnp.zeros((Np,), jnp.float32)
  ei3 = edge_index.reshape(2, E // 80, 80)

  agg1 = _make_agg(Np, H, E, with_deg=True)
  agg2 = _make_agg(Np, H, E, with_deg=False)

  p1, dpart = agg1(x, ei3, zeros2d, zeros1d)
  h = _combine(p1, dpart, x, W1l.T, W1r.T, b1, relu=True)
  p2 = agg2(h, ei3, zeros2d)
  P, Q = _combine(p2, dpart, h, W2l.T, W2r.T, b2, relu=False,
                  pq=(D1w[:, :H].T, D1w[:, H:].T))

  chunk = _NW * 112
  ELp = ((EL + chunk - 1) // chunk) * chunk
  pad = ELp - EL
  eli3 = jnp.concatenate(
      [edge_label_index, jnp.zeros((2, pad), jnp.int32)],
      axis=1).reshape(2, ELp // 112, 112)
  prow, qcol = _make_pair_gather(N, H, ELp)(P, Q, eli3)
  out = _decoder(prow, qcol, D1b, D2w.reshape(H), D2b)
  return out[:EL]


# speedup vs baseline: 1.8145x; 1.0051x over previous
import functools

import jax
import jax.numpy as jnp
from jax import lax
from jax.experimental import pallas as pl
from jax.experimental.pallas import tpu as pltpu
from jax.experimental.pallas import tpu_sc as plsc

_NC = 2
_NS = 16
_NW = _NC * _NS


_NBUF = 4


def _make_agg(Np, H, E, with_deg):
  e_per_w = E // _NW
  CH = 80
  assert e_per_w % CH == 0 and E % _NW == 0
  rows_per_tile = Np // _NS
  assert rows_per_tile % 8 == 0
  n_ch = e_per_w // CH
  n_grp = n_ch // _NBUF
  rem = n_ch - n_grp * _NBUF

  mesh = plsc.VectorSubcoreMesh(
      core_axis_name="c", subcore_axis_name="s",
      num_cores=_NC, num_subcores=_NS)

  out_type = [jax.ShapeDtypeStruct((_NC, Np, H), jnp.float32)]
  if with_deg:
    out_type.append(jax.ShapeDtypeStruct((_NC, Np), jnp.float32))

  scratch = (
      [pltpu.VMEM((CH,), jnp.int32) for _ in range(_NBUF)]
      + [pltpu.VMEM((CH,), jnp.int32) for _ in range(_NBUF)]
      + [pltpu.VMEM((CH, H), jnp.float32) for _ in range(_NBUF)]
      + [pltpu.VMEM_SHARED((Np, H), jnp.float32)]
      + [pltpu.SemaphoreType.DMA for _ in range(2 * _NBUF)]
  )
  if with_deg:
    scratch += (
        [pltpu.VMEM((CH,), jnp.float32),
         pltpu.VMEM_SHARED((Np,), jnp.float32)]
        + [pltpu.SemaphoreType.DMA for _ in range(_NBUF)]
    )

  def body(x_hbm, ei_hbm, z2_hbm, z1_hbm, out_hbm, deg_hbm, *rest):
    src_v = rest[:_NBUF]
    dst_v = rest[_NBUF:2 * _NBUF]
    rows_v = rest[2 * _NBUF:3 * _NBUF]
    acc_sh = rest[3 * _NBUF]
    gsem = rest[3 * _NBUF + 1:3 * _NBUF + 1 + _NBUF]
    ssem = rest[3 * _NBUF + 1 + _NBUF:3 * _NBUF + 1 + 2 * _NBUF]
    if with_deg:
      tail = rest[3 * _NBUF + 1 + 2 * _NBUF:]
      ones_v, deg_sh = tail[0], tail[1]
      dsem = tail[2:2 + _NBUF]

    cid = lax.axis_index("c")
    sid = lax.axis_index("s")
    wid = sid * _NC + cid
    r0 = sid * rows_per_tile
    pltpu.sync_copy(z2_hbm.at[pl.ds(r0, rows_per_tile)],
                    acc_sh.at[pl.ds(r0, rows_per_tile)])
    if with_deg:
      for i in range(CH // 16):
        ones_v[pl.ds(16 * i, 16)] = jnp.full((16,), 1.0, jnp.float32)

      @pl.when(sid == 0)
      def _():
        pltpu.sync_copy(z1_hbm, deg_sh)
    plsc.subcore_barrier()

    base = wid * n_ch

    def load_and_gather(b, c):
      r = base + c
      pltpu.sync_copy(ei_hbm.at[0, r], src_v[b])
      pltpu.sync_copy(ei_hbm.at[1, r], dst_v[b])
      pltpu.async_copy(x_hbm.at[src_v[b]], rows_v[b], gsem[b])

    def start_scatter(b):
      pltpu.async_copy(rows_v[b], acc_sh.at[dst_v[b]], ssem[b], add=True)
      if with_deg:
        pltpu.async_copy(ones_v, deg_sh.at[dst_v[b]], dsem[b], add=True)

    def wait_gather(b):
      pltpu.make_async_copy(x_hbm.at[src_v[b]], rows_v[b], gsem[b]).wait()

    def wait_scatter(b):
      pltpu.make_async_copy(rows_v[b], acc_sh.at[dst_v[b]], ssem[b]).wait()
      if with_deg:
        pltpu.make_async_copy(ones_v, deg_sh.at[dst_v[b]], dsem[b]).wait()

    for b in range(_NBUF):
      load_and_gather(b, b)

    def group(g, carry):
      for b in range(_NBUF):
        wait_gather(b)
        start_scatter(b)
      for b in range(_NBUF):
        wait_scatter(b)
        load_and_gather(b, (g + 1) * _NBUF + b)
      return carry

    lax.fori_loop(0, n_grp - 1, group, 0)
    for b in range(_NBUF):
      wait_gather(b)
      start_scatter(b)
    for b in range(_NBUF):
      wait_scatter(b)
    for r in range(rem):
      load_and_gather(r, n_grp * _NBUF + r)
      wait_gather(r)
      start_scatter(r)
      wait_scatter(r)

    plsc.subcore_barrier()
    pltpu.sync_copy(acc_sh.at[pl.ds(r0, rows_per_tile)],
                    out_hbm.at[cid, pl.ds(r0, rows_per_tile)])
    if with_deg:
      @pl.when(sid == 0)
      def _():
        pltpu.sync_copy(deg_sh, deg_hbm.at[cid])

  if with_deg:
    def body_wd(x_hbm, ei_hbm, z2_hbm, z1_hbm, out_hbm, deg_hbm, *rest):
      body(x_hbm, ei_hbm, z2_hbm, z1_hbm, out_hbm, deg_hbm, *rest)
    fn = pl.kernel(body_wd, out_type=out_type, mesh=mesh,
                   scratch_types=scratch)
    return lambda x, ei3, z2, z1: fn(x, ei3, z2, z1)
  else:
    def body_nd(x_hbm, ei_hbm, z2_hbm, out_hbm, *rest):
      body(x_hbm, ei_hbm, z2_hbm, None, out_hbm, None, *rest)
    fn = pl.kernel(body_nd, out_type=out_type[0], mesh=mesh,
                   scratch_types=scratch)
    return lambda x, ei3, z2: fn(x, ei3, z2)


def _combine(p, dpart, x, Wlt, Wrt, b, relu, pq=None):
  N, H = x.shape
  bm = 1024
  grid = (pl.cdiv(N, bm),)

  def compute_z(p_ref, d_ref, x_ref, wl_ref, wr_ref, b_ref):
    agg = p_ref[0] + p_ref[1]
    d = d_ref[0] + d_ref[1]
    scale = 1.0 / jnp.maximum(d, 1.0)
    aggn = agg * scale[:, None]
    acc = jnp.dot(aggn, wl_ref[...], preferred_element_type=jnp.float32)
    acc = acc + jnp.dot(x_ref[...], wr_ref[...],
                        preferred_element_type=jnp.float32)
    acc = acc + b_ref[...][None, :]
    if relu:
      acc = jnp.maximum(acc, 0.0)
    return acc

  base_specs = [
      pl.BlockSpec((_NC, bm, H), lambda i: (0, i, 0)),
      pl.BlockSpec((_NC, bm), lambda i: (0, i)),
      pl.BlockSpec((bm, H), lambda i: (i, 0)),
      pl.BlockSpec((H, H), lambda i: (0, 0)),
      pl.BlockSpec((H, H), lambda i: (0, 0)),
      pl.BlockSpec((H,), lambda i: (0,)),
  ]

  if pq is None:
    def body(p_ref, d_ref, x_ref, wl_ref, wr_ref, b_ref, o_ref):
      o_ref[...] = compute_z(p_ref, d_ref, x_ref, wl_ref, wr_ref, b_ref)

    return pl.pallas_call(
        body,
        grid=grid,
        in_specs=base_specs,
        out_specs=pl.BlockSpec((bm, H), lambda i: (i, 0)),
        out_shape=jax.ShapeDtypeStruct((N, H), jnp.float32),
    )(p, dpart, x, Wlt, Wrt, b)

  At, Bt = pq

  def body_pq(p_ref, d_ref, x_ref, wl_ref, wr_ref, b_ref, at_ref, bt_ref,
              po_ref, qo_ref):
    z = compute_z(p_ref, d_ref, x_ref, wl_ref, wr_ref, b_ref)
    po_ref[...] = jnp.dot(z, at_ref[...], preferred_element_type=jnp.float32)
    qo_ref[...] = jnp.dot(z, bt_ref[...], preferred_element_type=jnp.float32)

  return pl.pallas_call(
      body_pq,
      grid=grid,
      in_specs=base_specs + [
          pl.BlockSpec((H, H), lambda i: (0, 0)),
          pl.BlockSpec((H, H), lambda i: (0, 0)),
      ],
      out_specs=[
          pl.BlockSpec((bm, H), lambda i: (i, 0)),
          pl.BlockSpec((bm, H), lambda i: (i, 0)),
      ],
      out_shape=[
          jax.ShapeDtypeStruct((N, H), jnp.float32),
          jax.ShapeDtypeStruct((N, H), jnp.float32),
      ],
  )(p, dpart, x, Wlt, Wrt, b, At, Bt)


def _make_pair_gather(N, H, ELp):
  per_w = ELp // _NW
  CH = 112
  assert per_w % CH == 0 and ELp % _NW == 0
  n_ch = per_w // CH

  mesh = plsc.VectorSubcoreMesh(
      core_axis_name="c", subcore_axis_name="s",
      num_cores=_NC, num_subcores=_NS)

  NB = 8
  n_pairs = 2 * n_ch
  assert n_pairs % NB == 0
  n_grp = n_pairs // NB

  out_type = [
      jax.ShapeDtypeStruct((ELp, H), jnp.float32),
      jax.ShapeDtypeStruct((ELp, H), jnp.float32),
  ]
  scratch = (
      [pltpu.VMEM((CH,), jnp.int32) for _ in range(NB)]
      + [pltpu.VMEM((CH, H), jnp.float32) for _ in range(NB)]
      + [pltpu.SemaphoreType.DMA for _ in range(2 * NB)]
  )

  def body(p_hbm, q_hbm, idx_hbm, o1_hbm, o2_hbm, *rest):
    idx_v = rest[:NB]
    rows_v = rest[NB:2 * NB]
    gsem = rest[2 * NB:3 * NB]
    wsem = rest[3 * NB:4 * NB]
    cid = lax.axis_index("c")
    sid = lax.axis_index("s")
    wid = sid * _NC + cid
    base = wid * per_w
    base_r = wid * n_ch

    def chunk_idx(b, g):
      return g * (NB // 2) + b // 2

    def load_and_gather(b, g):
      c = chunk_idx(b, g)
      tab = p_hbm if b % 2 == 0 else q_hbm
      pltpu.sync_copy(idx_hbm.at[b % 2, base_r + c], idx_v[b])
      pltpu.async_copy(tab.at[idx_v[b]], rows_v[b], gsem[b])

    def wait_gather(b):
      tab = p_hbm if b % 2 == 0 else q_hbm
      pltpu.make_async_copy(tab.at[idx_v[b]], rows_v[b], gsem[b]).wait()

    def start_write(b, g):
      off = pl.multiple_of(base + chunk_idx(b, g) * CH, 8)
      dst_hbm = o1_hbm if b % 2 == 0 else o2_hbm
      pltpu.async_copy(rows_v[b], dst_hbm.at[pl.ds(off, CH)], wsem[b])

    def wait_write(b):
      pltpu.make_async_copy(
          rows_v[b], o1_hbm.at[pl.ds(0, CH)], wsem[b]).wait()

    for b in range(NB):
      load_and_gather(b, 0)

    def group(g, carry):
      for b in range(NB):
        wait_gather(b)
        start_write(b, g)
      for b in range(NB):
        wait_write(b)
        load_and_gather(b, g + 1)
      return carry

    lax.fori_loop(0, n_grp - 1, group, 0)
    for b in range(NB):
      wait_gather(b)
      start_write(b, n_grp - 1)
    for b in range(NB):
      wait_write(b)

  fn = pl.kernel(body, out_type=out_type, mesh=mesh, scratch_types=scratch)
  return fn


def _decoder(prow, qcol, d1b, d2, d2b):
  ELp, H = prow.shape
  bm = 2048
  assert ELp % bm == 0
  grid = (ELp // bm,)

  def body(zr_ref, zc_ref, bias_ref, d2_ref, d2b_ref, o_ref):
    t = zr_ref[...] + zc_ref[...] + bias_ref[...][None, :]
    t = jnp.maximum(t, 0.0)
    o_ref[...] = jnp.sum(t * d2_ref[...][None, :], axis=1) + d2b_ref[0]

  return pl.pallas_call(
      body,
      grid=grid,
      in_specs=[
          pl.BlockSpec((bm, H), lambda i: (i, 0)),
          pl.BlockSpec((bm, H), lambda i: (i, 0)),
          pl.BlockSpec((H,), lambda i: (0,)),
          pl.BlockSpec((H,), lambda i: (0,)),
          pl.BlockSpec(memory_space=pltpu.SMEM),
      ],
      out_specs=pl.BlockSpec((bm,), lambda i: (i,)),
      out_shape=jax.ShapeDtypeStruct((ELp,), jnp.float32),
  )(prow, qcol, d1b, d2, d2b)


def kernel(x, edge_index, edge_label_index,
           W1l, W1r, b1, W2l, W2r, b2, D1w, D1b, D2w, D2b):
  N, H = x.shape
  E = edge_index.shape[1]
  EL = edge_label_index.shape[1]

  Np = ((N + 1023) // 1024) * 1024
  zeros2d = jnp.zeros((Np, H), jnp.float32)
  zeros1d = jnp.zeros((Np,), jnp.float32)
  ei3 = edge_index.reshape(2, E // 80, 80)

  agg1 = _make_agg(Np, H, E, with_deg=True)
  agg2 = _make_agg(Np, H, E, with_deg=False)

  p1, dpart = agg1(x, ei3, zeros2d, zeros1d)
  h = _combine(p1, dpart, x, W1l.T, W1r.T, b1, relu=True)
  p2 = agg2(h, ei3, zeros2d)
  P, Q = _combine(p2, dpart, h, W2l.T, W2r.T, b2, relu=False,
                  pq=(D1w[:, :H].T, D1w[:, H:].T))

  chunk = _NW * 112
  ELp = ((EL + chunk - 1) // chunk) * chunk
  pad = ELp - EL
  eli3 = jnp.concatenate(
      [edge_label_index, jnp.zeros((2, pad), jnp.int32)],
      axis=1).reshape(2, ELp // 112, 112)
  prow, qcol = _make_pair_gather(N, H, ELp)(P, Q, eli3)
  out = _decoder(prow, qcol, D1b, D2w.reshape(H), D2b)
  return out[:EL]


# speedup vs baseline: 1.8509x; 1.0200x over previous
import functools

import jax
import jax.numpy as jnp
from jax import lax
from jax.experimental import pallas as pl
from jax.experimental.pallas import tpu as pltpu
from jax.experimental.pallas import tpu_sc as plsc

_NC = 2
_NS = 16
_NW = _NC * _NS


_NBUF = 4


def _make_agg(Np, H, E, with_deg):
  e_per_w = E // _NW
  CH = 80
  assert e_per_w % CH == 0 and E % _NW == 0
  rows_per_tile = Np // _NS
  assert rows_per_tile % 8 == 0
  n_ch = e_per_w // CH
  n_grp = n_ch // _NBUF
  rem = n_ch - n_grp * _NBUF

  mesh = plsc.VectorSubcoreMesh(
      core_axis_name="c", subcore_axis_name="s",
      num_cores=_NC, num_subcores=_NS)

  out_type = [jax.ShapeDtypeStruct((_NC, Np, H), jnp.float32)]
  if with_deg:
    out_type.append(jax.ShapeDtypeStruct((_NC, Np), jnp.float32))

  scratch = (
      [pltpu.VMEM((CH,), jnp.int32) for _ in range(_NBUF)]
      + [pltpu.VMEM((CH,), jnp.int32) for _ in range(_NBUF)]
      + [pltpu.VMEM((CH, H), jnp.float32) for _ in range(_NBUF)]
      + [pltpu.VMEM_SHARED((Np, H), jnp.float32)]
      + [pltpu.SemaphoreType.DMA for _ in range(2 * _NBUF)]
  )
  if with_deg:
    scratch += (
        [pltpu.VMEM((CH,), jnp.float32),
         pltpu.VMEM_SHARED((Np,), jnp.float32)]
        + [pltpu.SemaphoreType.DMA for _ in range(_NBUF)]
    )

  def body(x_hbm, ei_hbm, z2_hbm, z1_hbm, out_hbm, deg_hbm, *rest):
    src_v = rest[:_NBUF]
    dst_v = rest[_NBUF:2 * _NBUF]
    rows_v = rest[2 * _NBUF:3 * _NBUF]
    acc_sh = rest[3 * _NBUF]
    gsem = rest[3 * _NBUF + 1:3 * _NBUF + 1 + _NBUF]
    ssem = rest[3 * _NBUF + 1 + _NBUF:3 * _NBUF + 1 + 2 * _NBUF]
    if with_deg:
      tail = rest[3 * _NBUF + 1 + 2 * _NBUF:]
      ones_v, deg_sh = tail[0], tail[1]
      dsem = tail[2:2 + _NBUF]

    cid = lax.axis_index("c")
    sid = lax.axis_index("s")
    wid = sid * _NC + cid
    r0 = sid * rows_per_tile
    pltpu.sync_copy(z2_hbm.at[pl.ds(r0, rows_per_tile)],
                    acc_sh.at[pl.ds(r0, rows_per_tile)])
    if with_deg:
      for i in range(CH // 16):
        ones_v[pl.ds(16 * i, 16)] = jnp.full((16,), 1.0, jnp.float32)

      @pl.when(sid == 0)
      def _():
        pltpu.sync_copy(z1_hbm, deg_sh)
    plsc.subcore_barrier()

    base = wid * n_ch

    def load_and_gather(b, c):
      r = base + c
      pltpu.sync_copy(ei_hbm.at[0, r], src_v[b])
      pltpu.sync_copy(ei_hbm.at[1, r], dst_v[b])
      pltpu.async_copy(x_hbm.at[src_v[b]], rows_v[b], gsem[b])

    def start_scatter(b):
      pltpu.async_copy(rows_v[b], acc_sh.at[dst_v[b]], ssem[b], add=True)
      if with_deg:
        pltpu.async_copy(ones_v, deg_sh.at[dst_v[b]], dsem[b], add=True)

    def wait_gather(b):
      pltpu.make_async_copy(x_hbm.at[src_v[b]], rows_v[b], gsem[b]).wait()

    def wait_scatter(b):
      pltpu.make_async_copy(rows_v[b], acc_sh.at[dst_v[b]], ssem[b]).wait()
      if with_deg:
        pltpu.make_async_copy(ones_v, deg_sh.at[dst_v[b]], dsem[b]).wait()

    for b in range(_NBUF):
      load_and_gather(b, b)

    def group(g, carry):
      for b in range(_NBUF):
        wait_gather(b)
        start_scatter(b)
      for b in range(_NBUF):
        wait_scatter(b)
        load_and_gather(b, (g + 1) * _NBUF + b)
      return carry

    lax.fori_loop(0, n_grp - 1, group, 0)
    for b in range(_NBUF):
      wait_gather(b)
      start_scatter(b)
    for b in range(_NBUF):
      wait_scatter(b)
    for r in range(rem):
      load_and_gather(r, n_grp * _NBUF + r)
      wait_gather(r)
      start_scatter(r)
      wait_scatter(r)

    plsc.subcore_barrier()
    pltpu.sync_copy(acc_sh.at[pl.ds(r0, rows_per_tile)],
                    out_hbm.at[cid, pl.ds(r0, rows_per_tile)])
    if with_deg:
      @pl.when(sid == 0)
      def _():
        pltpu.sync_copy(deg_sh, deg_hbm.at[cid])

  if with_deg:
    def body_wd(x_hbm, ei_hbm, z2_hbm, z1_hbm, out_hbm, deg_hbm, *rest):
      body(x_hbm, ei_hbm, z2_hbm, z1_hbm, out_hbm, deg_hbm, *rest)
    fn = pl.kernel(body_wd, out_type=out_type, mesh=mesh,
                   scratch_types=scratch)
    return lambda x, ei3, z2, z1: fn(x, ei3, z2, z1)
  else:
    def body_nd(x_hbm, ei_hbm, z2_hbm, out_hbm, *rest):
      body(x_hbm, ei_hbm, z2_hbm, None, out_hbm, None, *rest)
    fn = pl.kernel(body_nd, out_type=out_type[0], mesh=mesh,
                   scratch_types=scratch)
    return lambda x, ei3, z2: fn(x, ei3, z2)


def _combine(p, dpart, x, Wlt, Wrt, b, relu, pq=None):
  N, H = x.shape
  bm = 1024
  grid = (pl.cdiv(N, bm),)

  def compute_z(p_ref, d_ref, x_ref, wl_ref, wr_ref, b_ref):
    agg = p_ref[0] + p_ref[1]
    d = d_ref[0] + d_ref[1]
    scale = 1.0 / jnp.maximum(d, 1.0)
    aggn = agg * scale[:, None]
    acc = jnp.dot(aggn, wl_ref[...], preferred_element_type=jnp.float32)
    acc = acc + jnp.dot(x_ref[...], wr_ref[...],
                        preferred_element_type=jnp.float32)
    acc = acc + b_ref[...][None, :]
    if relu:
      acc = jnp.maximum(acc, 0.0)
    return acc

  base_specs = [
      pl.BlockSpec((_NC, bm, H), lambda i: (0, i, 0)),
      pl.BlockSpec((_NC, bm), lambda i: (0, i)),
      pl.BlockSpec((bm, H), lambda i: (i, 0)),
      pl.BlockSpec((H, H), lambda i: (0, 0)),
      pl.BlockSpec((H, H), lambda i: (0, 0)),
      pl.BlockSpec((H,), lambda i: (0,)),
  ]

  if pq is None:
    def body(p_ref, d_ref, x_ref, wl_ref, wr_ref, b_ref, o_ref):
      o_ref[...] = compute_z(p_ref, d_ref, x_ref, wl_ref, wr_ref, b_ref)

    return pl.pallas_call(
        body,
        grid=grid,
        in_specs=base_specs,
        out_specs=pl.BlockSpec((bm, H), lambda i: (i, 0)),
        out_shape=jax.ShapeDtypeStruct((N, H), jnp.float32),
    )(p, dpart, x, Wlt, Wrt, b)

  At, Bt = pq

  def body_pq(p_ref, d_ref, x_ref, wl_ref, wr_ref, b_ref, at_ref, bt_ref,
              po_ref, qo_ref):
    z = compute_z(p_ref, d_ref, x_ref, wl_ref, wr_ref, b_ref)
    po_ref[...] = jnp.dot(z, at_ref[...], preferred_element_type=jnp.float32)
    qo_ref[...] = jnp.dot(z, bt_ref[...], preferred_element_type=jnp.float32)

  return pl.pallas_call(
      body_pq,
      grid=grid,
      in_specs=base_specs + [
          pl.BlockSpec((H, H), lambda i: (0, 0)),
          pl.BlockSpec((H, H), lambda i: (0, 0)),
      ],
      out_specs=[
          pl.BlockSpec((bm, H), lambda i: (i, 0)),
          pl.BlockSpec((bm, H), lambda i: (i, 0)),
      ],
      out_shape=[
          jax.ShapeDtypeStruct((N, H), jnp.float32),
          jax.ShapeDtypeStruct((N, H), jnp.float32),
      ],
  )(p, dpart, x, Wlt, Wrt, b, At, Bt)


def _make_pair_gather(N, H, ELp):
  per_w = ELp // _NW
  CH = 112
  assert per_w % CH == 0 and ELp % _NW == 0
  n_ch = per_w // CH

  mesh = plsc.VectorSubcoreMesh(
      core_axis_name="c", subcore_axis_name="s",
      num_cores=_NC, num_subcores=_NS)

  NB = 4
  n_grp = n_ch // NB
  assert n_ch == n_grp * NB

  out_type = jax.ShapeDtypeStruct((ELp, H), jnp.float32)
  scratch = (
      [pltpu.VMEM((CH,), jnp.int32) for _ in range(NB)]
      + [pltpu.VMEM((CH,), jnp.int32) for _ in range(NB)]
      + [pltpu.VMEM((CH, H), jnp.float32) for _ in range(NB)]
      + [pltpu.VMEM((CH, H), jnp.float32) for _ in range(NB)]
      + [pltpu.SemaphoreType.DMA for _ in range(3 * NB)]
  )

  def body(p_hbm, q_hbm, idx_hbm, o_hbm, *rest):
    idxp = rest[:NB]
    idxq = rest[NB:2 * NB]
    bufp = rest[2 * NB:3 * NB]
    bufq = rest[3 * NB:4 * NB]
    gpsem = rest[4 * NB:5 * NB]
    gqsem = rest[5 * NB:6 * NB]
    wsem = rest[6 * NB:7 * NB]
    cid = lax.axis_index("c")
    sid = lax.axis_index("s")
    wid = sid * _NC + cid
    base = wid * per_w
    base_r = wid * n_ch

    def load_and_gather(b, c):
      r = base_r + c
      pltpu.sync_copy(idx_hbm.at[0, r], idxp[b])
      pltpu.sync_copy(idx_hbm.at[1, r], idxq[b])
      pltpu.async_copy(p_hbm.at[idxp[b]], bufp[b], gpsem[b])
      pltpu.async_copy(q_hbm.at[idxq[b]], bufq[b], gqsem[b])

    def wait_gathers(b):
      pltpu.make_async_copy(p_hbm.at[idxp[b]], bufp[b], gpsem[b]).wait()
      pltpu.make_async_copy(q_hbm.at[idxq[b]], bufq[b], gqsem[b]).wait()

    def tec_add(b):
      def rows(r2, carry):
        r = r2 * 2
        for rr in range(2):
          for j in range(H // 16):
            sl = pl.ds(16 * j, 16)
            bufp[b][r + rr, sl] = bufp[b][r + rr, sl] + bufq[b][r + rr, sl]
        return carry
      lax.fori_loop(0, CH // 2, rows, 0)

    def start_write(b, c):
      off = pl.multiple_of(base + c * CH, 8)
      pltpu.async_copy(bufp[b], o_hbm.at[pl.ds(off, CH)], wsem[b])

    def wait_write(b):
      pltpu.make_async_copy(bufp[b], o_hbm.at[pl.ds(0, CH)], wsem[b]).wait()

    for b in range(NB):
      load_and_gather(b, b)

    def group(g, carry):
      for b in range(NB):
        wait_gathers(b)
        tec_add(b)
        start_write(b, g * NB + b)
      for b in range(NB):
        wait_write(b)
        load_and_gather(b, (g + 1) * NB + b)
      return carry

    lax.fori_loop(0, n_grp - 1, group, 0)
    for b in range(NB):
      wait_gathers(b)
      tec_add(b)
      start_write(b, (n_grp - 1) * NB + b)
    for b in range(NB):
      wait_write(b)

  fn = pl.kernel(body, out_type=out_type, mesh=mesh, scratch_types=scratch)
  return fn


def _decoder(s, d1b, d2, d2b):
  ELp, H = s.shape
  bm = 2048
  assert ELp % bm == 0
  grid = (ELp // bm,)

  def body(s_ref, bias_ref, d2_ref, d2b_ref, o_ref):
    t = s_ref[...] + bias_ref[...][None, :]
    t = jnp.maximum(t, 0.0)
    o_ref[...] = jnp.sum(t * d2_ref[...][None, :], axis=1) + d2b_ref[0]

  return pl.pallas_call(
      body,
      grid=grid,
      in_specs=[
          pl.BlockSpec((bm, H), lambda i: (i, 0)),
          pl.BlockSpec((H,), lambda i: (0,)),
          pl.BlockSpec((H,), lambda i: (0,)),
          pl.BlockSpec(memory_space=pltpu.SMEM),
      ],
      out_specs=pl.BlockSpec((bm,), lambda i: (i,)),
      out_shape=jax.ShapeDtypeStruct((ELp,), jnp.float32),
  )(s, d1b, d2, d2b)


def kernel(x, edge_index, edge_label_index,
           W1l, W1r, b1, W2l, W2r, b2, D1w, D1b, D2w, D2b):
  N, H = x.shape
  E = edge_index.shape[1]
  EL = edge_label_index.shape[1]

  Np = ((N + 1023) // 1024) * 1024
  zeros2d = jnp.zeros((Np, H), jnp.float32)
  zeros1d = jnp.zeros((Np,), jnp.float32)
  ei3 = edge_index.reshape(2, E // 80, 80)

  agg1 = _make_agg(Np, H, E, with_deg=True)
  agg2 = _make_agg(Np, H, E, with_deg=False)

  p1, dpart = agg1(x, ei3, zeros2d, zeros1d)
  h = _combine(p1, dpart, x, W1l.T, W1r.T, b1, relu=True)
  p2 = agg2(h, ei3, zeros2d)
  P, Q = _combine(p2, dpart, h, W2l.T, W2r.T, b2, relu=False,
                  pq=(D1w[:, :H].T, D1w[:, H:].T))

  chunk = _NW * 112
  ELp = ((EL + chunk - 1) // chunk) * chunk
  pad = ELp - EL
  eli3 = jnp.concatenate(
      [edge_label_index, jnp.zeros((2, pad), jnp.int32)],
      axis=1).reshape(2, ELp // 112, 112)
  s = _make_pair_gather(N, H, ELp)(P, Q, eli3)
  out = _decoder(s, D1b, D2w.reshape(H), D2b)
  return out[:EL]


# speedup vs baseline: 1.8815x; 1.0166x over previous
import functools

import jax
import jax.numpy as jnp
from jax import lax
from jax.experimental import pallas as pl
from jax.experimental.pallas import tpu as pltpu
from jax.experimental.pallas import tpu_sc as plsc

_NC = 2
_NS = 16
_NW = _NC * _NS


_NBUF = 4


def _make_agg(Np, H, E, with_deg):
  e_per_w = E // _NW
  CH = 80
  assert e_per_w % CH == 0 and E % _NW == 0
  rows_per_tile = Np // _NS
  assert rows_per_tile % 8 == 0
  n_ch = e_per_w // CH
  n_grp = n_ch // _NBUF
  rem = n_ch - n_grp * _NBUF

  mesh = plsc.VectorSubcoreMesh(
      core_axis_name="c", subcore_axis_name="s",
      num_cores=_NC, num_subcores=_NS)

  out_type = [jax.ShapeDtypeStruct((_NC, Np, H), jnp.float32)]
  if with_deg:
    out_type.append(jax.ShapeDtypeStruct((_NC, Np), jnp.float32))

  scratch = (
      [pltpu.VMEM((CH,), jnp.int32) for _ in range(_NBUF)]
      + [pltpu.VMEM((CH,), jnp.int32) for _ in range(_NBUF)]
      + [pltpu.VMEM((CH, H), jnp.float32) for _ in range(_NBUF)]
      + [pltpu.VMEM_SHARED((Np, H), jnp.float32)]
      + [pltpu.SemaphoreType.DMA for _ in range(2 * _NBUF)]
  )
  if with_deg:
    scratch += (
        [pltpu.VMEM((CH,), jnp.float32),
         pltpu.VMEM_SHARED((Np,), jnp.float32)]
        + [pltpu.SemaphoreType.DMA for _ in range(_NBUF)]
    )

  def body(x_hbm, ei_hbm, z2_hbm, z1_hbm, out_hbm, deg_hbm, *rest):
    src_v = rest[:_NBUF]
    dst_v = rest[_NBUF:2 * _NBUF]
    rows_v = rest[2 * _NBUF:3 * _NBUF]
    acc_sh = rest[3 * _NBUF]
    gsem = rest[3 * _NBUF + 1:3 * _NBUF + 1 + _NBUF]
    ssem = rest[3 * _NBUF + 1 + _NBUF:3 * _NBUF + 1 + 2 * _NBUF]
    if with_deg:
      tail = rest[3 * _NBUF + 1 + 2 * _NBUF:]
      ones_v, deg_sh = tail[0], tail[1]
      dsem = tail[2:2 + _NBUF]

    cid = lax.axis_index("c")
    sid = lax.axis_index("s")
    wid = sid * _NC + cid
    r0 = sid * rows_per_tile
    pltpu.sync_copy(z2_hbm.at[pl.ds(r0, rows_per_tile)],
                    acc_sh.at[pl.ds(r0, rows_per_tile)])
    if with_deg:
      for i in range(CH // 16):
        ones_v[pl.ds(16 * i, 16)] = jnp.full((16,), 1.0, jnp.float32)

      @pl.when(sid == 0)
      def _():
        pltpu.sync_copy(z1_hbm, deg_sh)
    plsc.subcore_barrier()

    base = wid * n_ch

    def load_and_gather(b, c):
      r = base + c
      pltpu.sync_copy(ei_hbm.at[0, r], src_v[b])
      pltpu.sync_copy(ei_hbm.at[1, r], dst_v[b])
      pltpu.async_copy(x_hbm.at[src_v[b]], rows_v[b], gsem[b])

    def start_scatter(b):
      pltpu.async_copy(rows_v[b], acc_sh.at[dst_v[b]], ssem[b], add=True)
      if with_deg:
        pltpu.async_copy(ones_v, deg_sh.at[dst_v[b]], dsem[b], add=True)

    def wait_gather(b):
      pltpu.make_async_copy(x_hbm.at[src_v[b]], rows_v[b], gsem[b]).wait()

    def wait_scatter(b):
      pltpu.make_async_copy(rows_v[b], acc_sh.at[dst_v[b]], ssem[b]).wait()
      if with_deg:
        pltpu.make_async_copy(ones_v, deg_sh.at[dst_v[b]], dsem[b]).wait()

    for b in range(_NBUF):
      load_and_gather(b, b)

    def group(g, carry):
      for b in range(_NBUF):
        wait_gather(b)
        start_scatter(b)
      for b in range(_NBUF):
        wait_scatter(b)
        load_and_gather(b, (g + 1) * _NBUF + b)
      return carry

    lax.fori_loop(0, n_grp - 1, group, 0)
    for b in range(_NBUF):
      wait_gather(b)
      start_scatter(b)
    for b in range(_NBUF):
      wait_scatter(b)
    for r in range(rem):
      load_and_gather(r, n_grp * _NBUF + r)
      wait_gather(r)
      start_scatter(r)
      wait_scatter(r)

    plsc.subcore_barrier()
    pltpu.sync_copy(acc_sh.at[pl.ds(r0, rows_per_tile)],
                    out_hbm.at[cid, pl.ds(r0, rows_per_tile)])
    if with_deg:
      @pl.when(sid == 0)
      def _():
        pltpu.sync_copy(deg_sh, deg_hbm.at[cid])

  if with_deg:
    def body_wd(x_hbm, ei_hbm, z2_hbm, z1_hbm, out_hbm, deg_hbm, *rest):
      body(x_hbm, ei_hbm, z2_hbm, z1_hbm, out_hbm, deg_hbm, *rest)
    fn = pl.kernel(body_wd, out_type=out_type, mesh=mesh,
                   scratch_types=scratch)
    return lambda x, ei3, z2, z1: fn(x, ei3, z2, z1)
  else:
    def body_nd(x_hbm, ei_hbm, z2_hbm, out_hbm, *rest):
      body(x_hbm, ei_hbm, z2_hbm, None, out_hbm, None, *rest)
    fn = pl.kernel(body_nd, out_type=out_type[0], mesh=mesh,
                   scratch_types=scratch)
    return lambda x, ei3, z2: fn(x, ei3, z2)


def _combine(p, dpart, x, Wlt, Wrt, b, relu, pq=None):
  N, H = x.shape
  bm = 1024
  grid = (pl.cdiv(N, bm),)

  def compute_z(p_ref, d_ref, x_ref, wl_ref, wr_ref, b_ref):
    agg = p_ref[0] + p_ref[1]
    d = d_ref[0] + d_ref[1]
    scale = 1.0 / jnp.maximum(d, 1.0)
    aggn = agg * scale[:, None]
    acc = jnp.dot(aggn, wl_ref[...], preferred_element_type=jnp.float32)
    acc = acc + jnp.dot(x_ref[...], wr_ref[...],
                        preferred_element_type=jnp.float32)
    acc = acc + b_ref[...][None, :]
    if relu:
      acc = jnp.maximum(acc, 0.0)
    return acc

  base_specs = [
      pl.BlockSpec((_NC, bm, H), lambda i: (0, i, 0)),
      pl.BlockSpec((_NC, bm), lambda i: (0, i)),
      pl.BlockSpec((bm, H), lambda i: (i, 0)),
      pl.BlockSpec((H, H), lambda i: (0, 0)),
      pl.BlockSpec((H, H), lambda i: (0, 0)),
      pl.BlockSpec((H,), lambda i: (0,)),
  ]

  if pq is None:
    def body(p_ref, d_ref, x_ref, wl_ref, wr_ref, b_ref, o_ref):
      o_ref[...] = compute_z(p_ref, d_ref, x_ref, wl_ref, wr_ref, b_ref)

    return pl.pallas_call(
        body,
        grid=grid,
        in_specs=base_specs,
        out_specs=pl.BlockSpec((bm, H), lambda i: (i, 0)),
        out_shape=jax.ShapeDtypeStruct((N, H), jnp.float32),
    )(p, dpart, x, Wlt, Wrt, b)

  At, Bt = pq

  def body_pq(p_ref, d_ref, x_ref, wl_ref, wr_ref, b_ref, at_ref, bt_ref,
              po_ref, qo_ref):
    z = compute_z(p_ref, d_ref, x_ref, wl_ref, wr_ref, b_ref)
    po_ref[...] = jnp.dot(z, at_ref[...], preferred_element_type=jnp.float32)
    qo_ref[...] = jnp.dot(z, bt_ref[...], preferred_element_type=jnp.float32)

  return pl.pallas_call(
      body_pq,
      grid=grid,
      in_specs=base_specs + [
          pl.BlockSpec((H, H), lambda i: (0, 0)),
          pl.BlockSpec((H, H), lambda i: (0, 0)),
      ],
      out_specs=[
          pl.BlockSpec((bm, H), lambda i: (i, 0)),
          pl.BlockSpec((bm, H), lambda i: (i, 0)),
      ],
      out_shape=[
          jax.ShapeDtypeStruct((N, H), jnp.float32),
          jax.ShapeDtypeStruct((N, H), jnp.float32),
      ],
  )(p, dpart, x, Wlt, Wrt, b, At, Bt)


def _make_pair_gather(N, H, ELp):
  per_w = ELp // _NW
  CH = 112
  assert per_w % CH == 0 and ELp % _NW == 0
  n_ch = per_w // CH

  mesh = plsc.VectorSubcoreMesh(
      core_axis_name="c", subcore_axis_name="s",
      num_cores=_NC, num_subcores=_NS)

  NB = 4
  n_grp = n_ch // NB
  assert n_ch == n_grp * NB

  out_type = jax.ShapeDtypeStruct((ELp, H), jnp.float32)
  scratch = (
      [pltpu.VMEM((CH,), jnp.int32) for _ in range(NB)]
      + [pltpu.VMEM((CH,), jnp.int32) for _ in range(NB)]
      + [pltpu.VMEM((CH, H), jnp.float32) for _ in range(NB)]
      + [pltpu.VMEM((CH, H), jnp.float32) for _ in range(NB)]
      + [pltpu.SemaphoreType.DMA for _ in range(3 * NB)]
  )

  def body(p_hbm, q_hbm, idx_hbm, o_hbm, *rest):
    idxp = rest[:NB]
    idxq = rest[NB:2 * NB]
    bufp = rest[2 * NB:3 * NB]
    bufq = rest[3 * NB:4 * NB]
    gpsem = rest[4 * NB:5 * NB]
    gqsem = rest[5 * NB:6 * NB]
    wsem = rest[6 * NB:7 * NB]
    cid = lax.axis_index("c")
    sid = lax.axis_index("s")
    wid = sid * _NC + cid
    base = wid * per_w
    base_r = wid * n_ch

    def load_and_gather(b, c):
      r = base_r + c
      pltpu.sync_copy(idx_hbm.at[0, r], idxp[b])
      pltpu.sync_copy(idx_hbm.at[1, r], idxq[b])
      pltpu.async_copy(p_hbm.at[idxp[b]], bufp[b], gpsem[b])
      pltpu.async_copy(q_hbm.at[idxq[b]], bufq[b], gqsem[b])

    def wait_gathers(b):
      pltpu.make_async_copy(p_hbm.at[idxp[b]], bufp[b], gpsem[b]).wait()
      pltpu.make_async_copy(q_hbm.at[idxq[b]], bufq[b], gqsem[b]).wait()

    def tec_add(b):
      def rows(r2, carry):
        r = r2 * 2
        for rr in range(2):
          for j in range(H // 16):
            sl = pl.ds(16 * j, 16)
            bufp[b][r + rr, sl] = bufp[b][r + rr, sl] + bufq[b][r + rr, sl]
        return carry
      lax.fori_loop(0, CH // 2, rows, 0)

    def start_write(b, c):
      off = pl.multiple_of(base + c * CH, 8)
      pltpu.async_copy(bufp[b], o_hbm.at[pl.ds(off, CH)], wsem[b])

    def wait_write(b):
      pltpu.make_async_copy(bufp[b], o_hbm.at[pl.ds(0, CH)], wsem[b]).wait()

    for b in range(NB):
      load_and_gather(b, b)

    def group(g, carry):
      for b in range(NB):
        wait_gathers(b)
        tec_add(b)
        start_write(b, g * NB + b)
      for b in range(NB):
        wait_write(b)
        load_and_gather(b, (g + 1) * NB + b)
      return carry

    lax.fori_loop(0, n_grp - 1, group, 0)
    for b in range(NB):
      wait_gathers(b)
      tec_add(b)
      start_write(b, (n_grp - 1) * NB + b)
    for b in range(NB):
      wait_write(b)

  fn = pl.kernel(body, out_type=out_type, mesh=mesh, scratch_types=scratch)
  return fn


def _decoder(s, d1b, d2, d2b):
  ELp, H = s.shape
  bm = 7168
  assert ELp % bm == 0
  grid = (ELp // bm,)

  def body(s_ref, bias_ref, d2_ref, d2b_ref, o_ref):
    t = s_ref[...] + bias_ref[...][None, :]
    t = jnp.maximum(t, 0.0)
    o_ref[...] = jnp.sum(t * d2_ref[...][None, :], axis=1) + d2b_ref[0]

  return pl.pallas_call(
      body,
      grid=grid,
      in_specs=[
          pl.BlockSpec((bm, H), lambda i: (i, 0)),
          pl.BlockSpec((H,), lambda i: (0,)),
          pl.BlockSpec((H,), lambda i: (0,)),
          pl.BlockSpec(memory_space=pltpu.SMEM),
      ],
      out_specs=pl.BlockSpec((bm,), lambda i: (i,)),
      out_shape=jax.ShapeDtypeStruct((ELp,), jnp.float32),
  )(s, d1b, d2, d2b)


def kernel(x, edge_index, edge_label_index,
           W1l, W1r, b1, W2l, W2r, b2, D1w, D1b, D2w, D2b):
  N, H = x.shape
  E = edge_index.shape[1]
  EL = edge_label_index.shape[1]

  Np = ((N + 1023) // 1024) * 1024
  zeros2d = jnp.zeros((Np, H), jnp.float32)
  zeros1d = jnp.zeros((Np,), jnp.float32)
  ei3 = edge_index.reshape(2, E // 80, 80)

  agg1 = _make_agg(Np, H, E, with_deg=True)
  agg2 = _make_agg(Np, H, E, with_deg=False)

  p1, dpart = agg1(x, ei3, zeros2d, zeros1d)
  h = _combine(p1, dpart, x, W1l.T, W1r.T, b1, relu=True)
  p2 = agg2(h, ei3, zeros2d)
  P, Q = _combine(p2, dpart, h, W2l.T, W2r.T, b2, relu=False,
                  pq=(D1w[:, :H].T, D1w[:, H:].T))

  chunk = _NW * 112
  ELp = ((EL + chunk - 1) // chunk) * chunk
  pad = ELp - EL
  eli3 = jnp.concatenate(
      [edge_label_index, jnp.zeros((2, pad), jnp.int32)],
      axis=1).reshape(2, ELp // 112, 112)
  s = _make_pair_gather(N, H, ELp)(P, Q, eli3)
  out = _decoder(s, D1b, D2w.reshape(H), D2b)
  return out[:EL]


# speedup vs baseline: 1.8964x; 1.0079x over previous
import functools

import jax
import jax.numpy as jnp
from jax import lax
from jax.experimental import pallas as pl
from jax.experimental.pallas import tpu as pltpu
from jax.experimental.pallas import tpu_sc as plsc

_NC = 2
_NS = 16
_NW = _NC * _NS


_NBUF = 4


def _make_agg(Np, H, E, with_deg):
  e_per_w = E // _NW
  CH = 80
  assert e_per_w % CH == 0 and E % _NW == 0
  rows_per_tile = Np // _NS
  assert rows_per_tile % 8 == 0
  n_ch = e_per_w // CH
  n_grp = n_ch // _NBUF
  rem = n_ch - n_grp * _NBUF

  mesh = plsc.VectorSubcoreMesh(
      core_axis_name="c", subcore_axis_name="s",
      num_cores=_NC, num_subcores=_NS)

  out_type = [jax.ShapeDtypeStruct((_NC, Np, H), jnp.float32)]
  if with_deg:
    out_type.append(jax.ShapeDtypeStruct((_NC, Np), jnp.float32))

  scratch = (
      [pltpu.VMEM((CH,), jnp.int32) for _ in range(_NBUF)]
      + [pltpu.VMEM((CH,), jnp.int32) for _ in range(_NBUF)]
      + [pltpu.VMEM((CH, H), jnp.float32) for _ in range(_NBUF)]
      + [pltpu.VMEM_SHARED((Np, H), jnp.float32)]
      + [pltpu.SemaphoreType.DMA for _ in range(2 * _NBUF)]
  )
  if with_deg:
    scratch += (
        [pltpu.VMEM((CH,), jnp.float32),
         pltpu.VMEM_SHARED((Np,), jnp.float32)]
        + [pltpu.SemaphoreType.DMA for _ in range(_NBUF)]
    )

  def body(x_hbm, ei_hbm, z2_hbm, z1_hbm, out_hbm, deg_hbm, *rest):
    src_v = rest[:_NBUF]
    dst_v = rest[_NBUF:2 * _NBUF]
    rows_v = rest[2 * _NBUF:3 * _NBUF]
    acc_sh = rest[3 * _NBUF]
    gsem = rest[3 * _NBUF + 1:3 * _NBUF + 1 + _NBUF]
    ssem = rest[3 * _NBUF + 1 + _NBUF:3 * _NBUF + 1 + 2 * _NBUF]
    if with_deg:
      tail = rest[3 * _NBUF + 1 + 2 * _NBUF:]
      ones_v, deg_sh = tail[0], tail[1]
      dsem = tail[2:2 + _NBUF]

    cid = lax.axis_index("c")
    sid = lax.axis_index("s")
    wid = sid * _NC + cid
    r0 = sid * rows_per_tile
    pltpu.sync_copy(z2_hbm.at[pl.ds(r0, rows_per_tile)],
                    acc_sh.at[pl.ds(r0, rows_per_tile)])
    if with_deg:
      for i in range(CH // 16):
        ones_v[pl.ds(16 * i, 16)] = jnp.full((16,), 1.0, jnp.float32)

      @pl.when(sid == 0)
      def _():
        pltpu.sync_copy(z1_hbm, deg_sh)
    plsc.subcore_barrier()

    base = wid * n_ch

    def load_and_gather(b, c):
      r = base + c
      pltpu.sync_copy(ei_hbm.at[0, r], src_v[b])
      pltpu.sync_copy(ei_hbm.at[1, r], dst_v[b])
      pltpu.async_copy(x_hbm.at[src_v[b]], rows_v[b], gsem[b])

    def start_scatter(b):
      pltpu.async_copy(rows_v[b], acc_sh.at[dst_v[b]], ssem[b], add=True)
      if with_deg:
        pltpu.async_copy(ones_v, deg_sh.at[dst_v[b]], dsem[b], add=True)

    def wait_gather(b):
      pltpu.make_async_copy(x_hbm.at[src_v[b]], rows_v[b], gsem[b]).wait()

    def wait_scatter(b):
      pltpu.make_async_copy(rows_v[b], acc_sh.at[dst_v[b]], ssem[b]).wait()
      if with_deg:
        pltpu.make_async_copy(ones_v, deg_sh.at[dst_v[b]], dsem[b]).wait()

    for b in range(_NBUF):
      load_and_gather(b, b)

    def group(g, carry):
      for b in range(_NBUF):
        wait_gather(b)
        start_scatter(b)
      for b in range(_NBUF):
        wait_scatter(b)
        load_and_gather(b, (g + 1) * _NBUF + b)
      return carry

    lax.fori_loop(0, n_grp - 1, group, 0)
    for b in range(_NBUF):
      wait_gather(b)
      start_scatter(b)
    for b in range(_NBUF):
      wait_scatter(b)
    for r in range(rem):
      load_and_gather(r, n_grp * _NBUF + r)
      wait_gather(r)
      start_scatter(r)
      wait_scatter(r)

    plsc.subcore_barrier()
    pltpu.sync_copy(acc_sh.at[pl.ds(r0, rows_per_tile)],
                    out_hbm.at[cid, pl.ds(r0, rows_per_tile)])
    if with_deg:
      @pl.when(sid == 0)
      def _():
        pltpu.sync_copy(deg_sh, deg_hbm.at[cid])

  if with_deg:
    def body_wd(x_hbm, ei_hbm, z2_hbm, z1_hbm, out_hbm, deg_hbm, *rest):
      body(x_hbm, ei_hbm, z2_hbm, z1_hbm, out_hbm, deg_hbm, *rest)
    fn = pl.kernel(body_wd, out_type=out_type, mesh=mesh,
                   scratch_types=scratch)
    return lambda x, ei3, z2, z1: fn(x, ei3, z2, z1)
  else:
    def body_nd(x_hbm, ei_hbm, z2_hbm, out_hbm, *rest):
      body(x_hbm, ei_hbm, z2_hbm, None, out_hbm, None, *rest)
    fn = pl.kernel(body_nd, out_type=out_type[0], mesh=mesh,
                   scratch_types=scratch)
    return lambda x, ei3, z2: fn(x, ei3, z2)


def _combine(p, dpart, x, Wlt, Wrt, b, relu, pq=None):
  N, H = x.shape
  bm = 2048
  grid = (pl.cdiv(N, bm),)

  def compute_z(p_ref, d_ref, x_ref, wl_ref, wr_ref, b_ref):
    agg = p_ref[0] + p_ref[1]
    d = d_ref[0] + d_ref[1]
    scale = 1.0 / jnp.maximum(d, 1.0)
    aggn = agg * scale[:, None]
    acc = jnp.dot(aggn, wl_ref[...], preferred_element_type=jnp.float32)
    acc = acc + jnp.dot(x_ref[...], wr_ref[...],
                        preferred_element_type=jnp.float32)
    acc = acc + b_ref[...][None, :]
    if relu:
      acc = jnp.maximum(acc, 0.0)
    return acc

  base_specs = [
      pl.BlockSpec((_NC, bm, H), lambda i: (0, i, 0)),
      pl.BlockSpec((_NC, bm), lambda i: (0, i)),
      pl.BlockSpec((bm, H), lambda i: (i, 0)),
      pl.BlockSpec((H, H), lambda i: (0, 0)),
      pl.BlockSpec((H, H), lambda i: (0, 0)),
      pl.BlockSpec((H,), lambda i: (0,)),
  ]

  if pq is None:
    def body(p_ref, d_ref, x_ref, wl_ref, wr_ref, b_ref, o_ref):
      o_ref[...] = compute_z(p_ref, d_ref, x_ref, wl_ref, wr_ref, b_ref)

    return pl.pallas_call(
        body,
        grid=grid,
        in_specs=base_specs,
        out_specs=pl.BlockSpec((bm, H), lambda i: (i, 0)),
        out_shape=jax.ShapeDtypeStruct((N, H), jnp.float32),
    )(p, dpart, x, Wlt, Wrt, b)

  At, Bt = pq

  def body_pq(p_ref, d_ref, x_ref, wl_ref, wr_ref, b_ref, at_ref, bt_ref,
              po_ref, qo_ref):
    z = compute_z(p_ref, d_ref, x_ref, wl_ref, wr_ref, b_ref)
    po_ref[...] = jnp.dot(z, at_ref[...], preferred_element_type=jnp.float32)
    qo_ref[...] = jnp.dot(z, bt_ref[...], preferred_element_type=jnp.float32)

  return pl.pallas_call(
      body_pq,
      grid=grid,
      in_specs=base_specs + [
          pl.BlockSpec((H, H), lambda i: (0, 0)),
          pl.BlockSpec((H, H), lambda i: (0, 0)),
      ],
      out_specs=[
          pl.BlockSpec((bm, H), lambda i: (i, 0)),
          pl.BlockSpec((bm, H), lambda i: (i, 0)),
      ],
      out_shape=[
          jax.ShapeDtypeStruct((N, H), jnp.float32),
          jax.ShapeDtypeStruct((N, H), jnp.float32),
      ],
  )(p, dpart, x, Wlt, Wrt, b, At, Bt)


def _make_pair_gather(N, H, ELp):
  per_w = ELp // _NW
  CH = 112
  assert per_w % CH == 0 and ELp % _NW == 0
  n_ch = per_w // CH

  mesh = plsc.VectorSubcoreMesh(
      core_axis_name="c", subcore_axis_name="s",
      num_cores=_NC, num_subcores=_NS)

  NB = 4
  n_grp = n_ch // NB
  assert n_ch == n_grp * NB

  out_type = jax.ShapeDtypeStruct((ELp, H), jnp.float32)
  scratch = (
      [pltpu.VMEM((CH,), jnp.int32) for _ in range(NB)]
      + [pltpu.VMEM((CH,), jnp.int32) for _ in range(NB)]
      + [pltpu.VMEM((CH, H), jnp.float32) for _ in range(NB)]
      + [pltpu.VMEM((CH, H), jnp.float32) for _ in range(NB)]
      + [pltpu.SemaphoreType.DMA for _ in range(3 * NB)]
  )

  def body(p_hbm, q_hbm, idx_hbm, o_hbm, *rest):
    idxp = rest[:NB]
    idxq = rest[NB:2 * NB]
    bufp = rest[2 * NB:3 * NB]
    bufq = rest[3 * NB:4 * NB]
    gpsem = rest[4 * NB:5 * NB]
    gqsem = rest[5 * NB:6 * NB]
    wsem = rest[6 * NB:7 * NB]
    cid = lax.axis_index("c")
    sid = lax.axis_index("s")
    wid = sid * _NC + cid
    base = wid * per_w
    base_r = wid * n_ch

    def load_and_gather(b, c):
      r = base_r + c
      pltpu.sync_copy(idx_hbm.at[0, r], idxp[b])
      pltpu.sync_copy(idx_hbm.at[1, r], idxq[b])
      pltpu.async_copy(p_hbm.at[idxp[b]], bufp[b], gpsem[b])
      pltpu.async_copy(q_hbm.at[idxq[b]], bufq[b], gqsem[b])

    def wait_gathers(b):
      pltpu.make_async_copy(p_hbm.at[idxp[b]], bufp[b], gpsem[b]).wait()
      pltpu.make_async_copy(q_hbm.at[idxq[b]], bufq[b], gqsem[b]).wait()

    def tec_add(b):
      def rows(r2, carry):
        r = r2 * 2
        for rr in range(2):
          for j in range(H // 16):
            sl = pl.ds(16 * j, 16)
            bufp[b][r + rr, sl] = bufp[b][r + rr, sl] + bufq[b][r + rr, sl]
        return carry
      lax.fori_loop(0, CH // 2, rows, 0)

    def start_write(b, c):
      off = pl.multiple_of(base + c * CH, 8)
      pltpu.async_copy(bufp[b], o_hbm.at[pl.ds(off, CH)], wsem[b])

    def wait_write(b):
      pltpu.make_async_copy(bufp[b], o_hbm.at[pl.ds(0, CH)], wsem[b]).wait()

    for b in range(NB):
      load_and_gather(b, b)

    def group(g, carry):
      for b in range(NB):
        wait_gathers(b)
        tec_add(b)
        start_write(b, g * NB + b)
      for b in range(NB):
        wait_write(b)
        load_and_gather(b, (g + 1) * NB + b)
      return carry

    lax.fori_loop(0, n_grp - 1, group, 0)
    for b in range(NB):
      wait_gathers(b)
      tec_add(b)
      start_write(b, (n_grp - 1) * NB + b)
    for b in range(NB):
      wait_write(b)

  fn = pl.kernel(body, out_type=out_type, mesh=mesh, scratch_types=scratch)
  return fn


def _decoder(s, d1b, d2, d2b):
  ELp, H = s.shape
  bm = 7168
  assert ELp % bm == 0
  grid = (ELp // bm,)

  def body(s_ref, bias_ref, d2_ref, d2b_ref, o_ref):
    t = s_ref[...] + bias_ref[...][None, :]
    t = jnp.maximum(t, 0.0)
    o_ref[...] = jnp.sum(t * d2_ref[...][None, :], axis=1) + d2b_ref[0]

  return pl.pallas_call(
      body,
      grid=grid,
      in_specs=[
          pl.BlockSpec((bm, H), lambda i: (i, 0)),
          pl.BlockSpec((H,), lambda i: (0,)),
          pl.BlockSpec((H,), lambda i: (0,)),
          pl.BlockSpec(memory_space=pltpu.SMEM),
      ],
      out_specs=pl.BlockSpec((bm,), lambda i: (i,)),
      out_shape=jax.ShapeDtypeStruct((ELp,), jnp.float32),
  )(s, d1b, d2, d2b)


def kernel(x, edge_index, edge_label_index,
           W1l, W1r, b1, W2l, W2r, b2, D1w, D1b, D2w, D2b):
  N, H = x.shape
  E = edge_index.shape[1]
  EL = edge_label_index.shape[1]

  Np = ((N + 1023) // 1024) * 1024
  zeros2d = jnp.zeros((Np, H), jnp.float32)
  zeros1d = jnp.zeros((Np,), jnp.float32)
  ei3 = edge_index.reshape(2, E // 80, 80)

  agg1 = _make_agg(Np, H, E, with_deg=True)
  agg2 = _make_agg(Np, H, E, with_deg=False)

  p1, dpart = agg1(x, ei3, zeros2d, zeros1d)
  h = _combine(p1, dpart, x, W1l.T, W1r.T, b1, relu=True)
  p2 = agg2(h, ei3, zeros2d)
  P, Q = _combine(p2, dpart, h, W2l.T, W2r.T, b2, relu=False,
                  pq=(D1w[:, :H].T, D1w[:, H:].T))

  chunk = _NW * 112
  ELp = ((EL + chunk - 1) // chunk) * chunk
  pad = ELp - EL
  eli3 = jnp.concatenate(
      [edge_label_index, jnp.zeros((2, pad), jnp.int32)],
      axis=1).reshape(2, ELp // 112, 112)
  s = _make_pair_gather(N, H, ELp)(P, Q, eli3)
  out = _decoder(s, D1b, D2w.reshape(H), D2b)
  return out[:EL]


# speedup vs baseline: 1.8979x; 1.0008x over previous
import jax
import jax.numpy as jnp
from jax import lax
from jax.experimental import pallas as pl
from jax.experimental.pallas import tpu as pltpu
from jax.experimental.pallas import tpu_sc as plsc

_NC = 2
_NS = 16
_NW = _NC * _NS


_NBUF = 4


def _make_agg(Np, H, E, with_deg):
  e_per_w = E // _NW
  CH = 80
  assert e_per_w % CH == 0 and E % _NW == 0
  rows_per_tile = Np // _NS
  assert rows_per_tile % 8 == 0
  n_ch = e_per_w // CH
  n_grp = n_ch // _NBUF
  rem = n_ch - n_grp * _NBUF

  mesh = plsc.VectorSubcoreMesh(
      core_axis_name="c", subcore_axis_name="s",
      num_cores=_NC, num_subcores=_NS)

  out_type = [jax.ShapeDtypeStruct((_NC, Np, H), jnp.float32)]
  if with_deg:
    out_type.append(jax.ShapeDtypeStruct((_NC, Np), jnp.float32))

  scratch = (
      [pltpu.VMEM((CH,), jnp.int32) for _ in range(_NBUF)]
      + [pltpu.VMEM((CH,), jnp.int32) for _ in range(_NBUF)]
      + [pltpu.VMEM((CH, H), jnp.float32) for _ in range(_NBUF)]
      + [pltpu.VMEM_SHARED((Np, H), jnp.float32)]
      + [pltpu.SemaphoreType.DMA for _ in range(2 * _NBUF)]
  )
  if with_deg:
    scratch += (
        [pltpu.VMEM((CH,), jnp.float32),
         pltpu.VMEM_SHARED((Np,), jnp.float32)]
        + [pltpu.SemaphoreType.DMA for _ in range(_NBUF)]
    )

  def body(x_hbm, ei_hbm, z2_hbm, z1_hbm, out_hbm, deg_hbm, *rest):
    src_v = rest[:_NBUF]
    dst_v = rest[_NBUF:2 * _NBUF]
    rows_v = rest[2 * _NBUF:3 * _NBUF]
    acc_sh = rest[3 * _NBUF]
    gsem = rest[3 * _NBUF + 1:3 * _NBUF + 1 + _NBUF]
    ssem = rest[3 * _NBUF + 1 + _NBUF:3 * _NBUF + 1 + 2 * _NBUF]
    if with_deg:
      tail = rest[3 * _NBUF + 1 + 2 * _NBUF:]
      ones_v, deg_sh = tail[0], tail[1]
      dsem = tail[2:2 + _NBUF]

    cid = lax.axis_index("c")
    sid = lax.axis_index("s")
    wid = sid * _NC + cid
    r0 = sid * rows_per_tile
    pltpu.sync_copy(z2_hbm.at[pl.ds(r0, rows_per_tile)],
                    acc_sh.at[pl.ds(r0, rows_per_tile)])
    if with_deg:
      for i in range(CH // 16):
        ones_v[pl.ds(16 * i, 16)] = jnp.full((16,), 1.0, jnp.float32)

      @pl.when(sid == 0)
      def _():
        pltpu.sync_copy(z1_hbm, deg_sh)
    plsc.subcore_barrier()

    base = wid * n_ch

    def load_and_gather(b, c):
      r = base + c
      pltpu.sync_copy(ei_hbm.at[0, r], src_v[b])
      pltpu.sync_copy(ei_hbm.at[1, r], dst_v[b])
      pltpu.async_copy(x_hbm.at[src_v[b]], rows_v[b], gsem[b])

    def start_scatter(b):
      pltpu.async_copy(rows_v[b], acc_sh.at[dst_v[b]], ssem[b], add=True)
      if with_deg:
        pltpu.async_copy(ones_v, deg_sh.at[dst_v[b]], dsem[b], add=True)

    def wait_gather(b):
      pltpu.make_async_copy(x_hbm.at[src_v[b]], rows_v[b], gsem[b]).wait()

    def wait_scatter(b):
      pltpu.make_async_copy(rows_v[b], acc_sh.at[dst_v[b]], ssem[b]).wait()
      if with_deg:
        pltpu.make_async_copy(ones_v, deg_sh.at[dst_v[b]], dsem[b]).wait()

    for b in range(_NBUF):
      load_and_gather(b, b)

    def group(g, carry):
      for b in range(_NBUF):
        wait_gather(b)
        start_scatter(b)
      for b in range(_NBUF):
        wait_scatter(b)
        load_and_gather(b, (g + 1) * _NBUF + b)
      return carry

    lax.fori_loop(0, n_grp - 1, group, 0)
    for b in range(_NBUF):
      wait_gather(b)
      start_scatter(b)
    for b in range(_NBUF):
      wait_scatter(b)
    for r in range(rem):
      load_and_gather(r, n_grp * _NBUF + r)
      wait_gather(r)
      start_scatter(r)
      wait_scatter(r)

    plsc.subcore_barrier()
    pltpu.sync_copy(acc_sh.at[pl.ds(r0, rows_per_tile)],
                    out_hbm.at[cid, pl.ds(r0, rows_per_tile)])
    if with_deg:
      @pl.when(sid == 0)
      def _():
        pltpu.sync_copy(deg_sh, deg_hbm.at[cid])

  if with_deg:
    def body_wd(x_hbm, ei_hbm, z2_hbm, z1_hbm, out_hbm, deg_hbm, *rest):
      body(x_hbm, ei_hbm, z2_hbm, z1_hbm, out_hbm, deg_hbm, *rest)
    fn = pl.kernel(body_wd, out_type=out_type, mesh=mesh,
                   scratch_types=scratch)
    return lambda x, ei3, z2, z1: fn(x, ei3, z2, z1)
  else:
    def body_nd(x_hbm, ei_hbm, z2_hbm, out_hbm, *rest):
      body(x_hbm, ei_hbm, z2_hbm, None, out_hbm, None, *rest)
    fn = pl.kernel(body_nd, out_type=out_type[0], mesh=mesh,
                   scratch_types=scratch)
    return lambda x, ei3, z2: fn(x, ei3, z2)


def _combine(p, dpart, x, Wlt, Wrt, b, relu, pq=None):
  N, H = x.shape
  bm = 2048
  grid = (pl.cdiv(N, bm),)

  def compute_z(p_ref, d_ref, x_ref, wl_ref, wr_ref, b_ref):
    agg = p_ref[0] + p_ref[1]
    d = d_ref[0] + d_ref[1]
    scale = 1.0 / jnp.maximum(d, 1.0)
    aggn = agg * scale[:, None]
    acc = jnp.dot(aggn, wl_ref[...], preferred_element_type=jnp.float32)
    acc = acc + jnp.dot(x_ref[...], wr_ref[...],
                        preferred_element_type=jnp.float32)
    acc = acc + b_ref[...][None, :]
    if relu:
      acc = jnp.maximum(acc, 0.0)
    return acc

  base_specs = [
      pl.BlockSpec((_NC, bm, H), lambda i: (0, i, 0)),
      pl.BlockSpec((_NC, bm), lambda i: (0, i)),
      pl.BlockSpec((bm, H), lambda i: (i, 0)),
      pl.BlockSpec((H, H), lambda i: (0, 0)),
      pl.BlockSpec((H, H), lambda i: (0, 0)),
      pl.BlockSpec((H,), lambda i: (0,)),
  ]

  if pq is None:
    def body(p_ref, d_ref, x_ref, wl_ref, wr_ref, b_ref, o_ref):
      o_ref[...] = compute_z(p_ref, d_ref, x_ref, wl_ref, wr_ref, b_ref)

    return pl.pallas_call(
        body,
        grid=grid,
        in_specs=base_specs,
        out_specs=pl.BlockSpec((bm, H), lambda i: (i, 0)),
        out_shape=jax.ShapeDtypeStruct((N, H), jnp.float32),
    )(p, dpart, x, Wlt, Wrt, b)

  At, Bt = pq

  def body_pq(p_ref, d_ref, x_ref, wl_ref, wr_ref, b_ref, at_ref, bt_ref,
              po_ref, qo_ref):
    z = compute_z(p_ref, d_ref, x_ref, wl_ref, wr_ref, b_ref)
    po_ref[...] = jnp.dot(z, at_ref[...], preferred_element_type=jnp.float32)
    qo_ref[...] = jnp.dot(z, bt_ref[...], preferred_element_type=jnp.float32)

  return pl.pallas_call(
      body_pq,
      grid=grid,
      in_specs=base_specs + [
          pl.BlockSpec((H, H), lambda i: (0, 0)),
          pl.BlockSpec((H, H), lambda i: (0, 0)),
      ],
      out_specs=[
          pl.BlockSpec((bm, H), lambda i: (i, 0)),
          pl.BlockSpec((bm, H), lambda i: (i, 0)),
      ],
      out_shape=[
          jax.ShapeDtypeStruct((N, H), jnp.float32),
          jax.ShapeDtypeStruct((N, H), jnp.float32),
      ],
  )(p, dpart, x, Wlt, Wrt, b, At, Bt)


def _make_pair_gather(N, H, ELp):
  per_w = ELp // _NW
  CH = 112
  assert per_w % CH == 0 and ELp % _NW == 0
  n_ch = per_w // CH

  mesh = plsc.VectorSubcoreMesh(
      core_axis_name="c", subcore_axis_name="s",
      num_cores=_NC, num_subcores=_NS)

  NB = 4
  n_grp = n_ch // NB
  assert n_ch == n_grp * NB

  out_type = jax.ShapeDtypeStruct((ELp, H), jnp.float32)
  scratch = (
      [pltpu.VMEM((CH,), jnp.int32) for _ in range(NB)]
      + [pltpu.VMEM((CH,), jnp.int32) for _ in range(NB)]
      + [pltpu.VMEM((CH, H), jnp.float32) for _ in range(NB)]
      + [pltpu.VMEM((CH, H), jnp.float32) for _ in range(NB)]
      + [pltpu.SemaphoreType.DMA for _ in range(3 * NB)]
  )

  def body(p_hbm, q_hbm, idx_hbm, o_hbm, *rest):
    idxp = rest[:NB]
    idxq = rest[NB:2 * NB]
    bufp = rest[2 * NB:3 * NB]
    bufq = rest[3 * NB:4 * NB]
    gpsem = rest[4 * NB:5 * NB]
    gqsem = rest[5 * NB:6 * NB]
    wsem = rest[6 * NB:7 * NB]
    cid = lax.axis_index("c")
    sid = lax.axis_index("s")
    wid = sid * _NC + cid
    base = wid * per_w
    base_r = wid * n_ch

    def load_and_gather(b, c):
      r = base_r + c
      pltpu.sync_copy(idx_hbm.at[0, r], idxp[b])
      pltpu.sync_copy(idx_hbm.at[1, r], idxq[b])
      pltpu.async_copy(p_hbm.at[idxp[b]], bufp[b], gpsem[b])
      pltpu.async_copy(q_hbm.at[idxq[b]], bufq[b], gqsem[b])

    def wait_gathers(b):
      pltpu.make_async_copy(p_hbm.at[idxp[b]], bufp[b], gpsem[b]).wait()
      pltpu.make_async_copy(q_hbm.at[idxq[b]], bufq[b], gqsem[b]).wait()

    def tec_add(b):
      def rows(r2, carry):
        r = r2 * 2
        for rr in range(2):
          for j in range(H // 16):
            sl = pl.ds(16 * j, 16)
            bufp[b][r + rr, sl] = bufp[b][r + rr, sl] + bufq[b][r + rr, sl]
        return carry
      lax.fori_loop(0, CH // 2, rows, 0)

    def start_write(b, c):
      off = pl.multiple_of(base + c * CH, 8)
      pltpu.async_copy(bufp[b], o_hbm.at[pl.ds(off, CH)], wsem[b])

    def wait_write(b):
      pltpu.make_async_copy(bufp[b], o_hbm.at[pl.ds(0, CH)], wsem[b]).wait()

    for b in range(NB):
      load_and_gather(b, b)

    def group(g, carry):
      for b in range(NB):
        wait_gathers(b)
        tec_add(b)
        start_write(b, g * NB + b)
      for b in range(NB):
        wait_write(b)
        load_and_gather(b, (g + 1) * NB + b)
      return carry

    lax.fori_loop(0, n_grp - 1, group, 0)
    for b in range(NB):
      wait_gathers(b)
      tec_add(b)
      start_write(b, (n_grp - 1) * NB + b)
    for b in range(NB):
      wait_write(b)

  fn = pl.kernel(body, out_type=out_type, mesh=mesh, scratch_types=scratch)
  return fn


def _decoder(s, d1b, d2, d2b):
  ELp, H = s.shape
  bm = 7168
  assert ELp % bm == 0
  grid = (ELp // bm,)

  def body(s_ref, bias_ref, d2_ref, d2b_ref, o_ref):
    t = s_ref[...] + bias_ref[...][None, :]
    t = jnp.maximum(t, 0.0)
    o_ref[...] = jnp.sum(t * d2_ref[...][None, :], axis=1) + d2b_ref[0]

  return pl.pallas_call(
      body,
      grid=grid,
      in_specs=[
          pl.BlockSpec((bm, H), lambda i: (i, 0)),
          pl.BlockSpec((H,), lambda i: (0,)),
          pl.BlockSpec((H,), lambda i: (0,)),
          pl.BlockSpec(memory_space=pltpu.SMEM),
      ],
      out_specs=pl.BlockSpec((bm,), lambda i: (i,)),
      out_shape=jax.ShapeDtypeStruct((ELp,), jnp.float32),
  )(s, d1b, d2, d2b)


def kernel(x, edge_index, edge_label_index,
           W1l, W1r, b1, W2l, W2r, b2, D1w, D1b, D2w, D2b):
  N, H = x.shape
  E = edge_index.shape[1]
  EL = edge_label_index.shape[1]

  Np = ((N + 1023) // 1024) * 1024
  zeros2d = jnp.zeros((Np, H), jnp.float32)
  zeros1d = jnp.zeros((Np,), jnp.float32)
  ei3 = edge_index.reshape(2, E // 80, 80)

  agg1 = _make_agg(Np, H, E, with_deg=True)
  agg2 = _make_agg(Np, H, E, with_deg=False)

  p1, dpart = agg1(x, ei3, zeros2d, zeros1d)
  h = _combine(p1, dpart, x, W1l.T, W1r.T, b1, relu=True)
  p2 = agg2(h, ei3, zeros2d)
  P, Q = _combine(p2, dpart, h, W2l.T, W2r.T, b2, relu=False,
                  pq=(D1w[:, :H].T, D1w[:, H:].T))

  chunk = _NW * 112
  ELp = ((EL + chunk - 1) // chunk) * chunk
  pad = ELp - EL
  eli3 = jnp.concatenate(
      [edge_label_index, jnp.zeros((2, pad), jnp.int32)],
      axis=1).reshape(2, ELp // 112, 112)
  s = _make_pair_gather(N, H, ELp)(P, Q, eli3)
  out = _decoder(s, D1b, D2w.reshape(H), D2b)
  return out[:EL]
